# Initial kernel scaffold; baseline (speedup 1.0000x reference)
import jax
import jax.numpy as jnp
from jax.experimental import pallas as pl


def kernel(nodeblocks, x, W1, b1, W2, b2):
    raise NotImplementedError("write your pallas kernel here")


# speedup vs baseline: 22.8492x; 22.8492x over previous
import functools

import jax
import jax.numpy as jnp
from jax import lax
from jax.experimental import pallas as pl
from jax.experimental.pallas import tpu as pltpu
from jax.experimental.pallas import tpu_sc as plsc

N = 10000
E = 320000
DF = 128
DC = 40
EPS = 1e-5

NC, NS = 2, 16
NW = NC * NS
EW = E // NW
C = 80
K = EW // C
KD = 2 * EW // C
S = 10240
ZCH = 2 * S // NS
NP_ = 10112
RP = NP_ // NS

_mesh = plsc.VectorSubcoreMesh(core_axis_name="c", subcore_axis_name="s")


def _make_deg_kernel():
    @functools.partial(
        pl.kernel,
        out_type=jax.ShapeDtypeStruct((NC, 1, 2 * S), jnp.float32),
        mesh=_mesh,
        scratch_types=[
            pltpu.VMEM((KD, C), jnp.int32),
            pltpu.VMEM((C,), jnp.float32),
            pltpu.VMEM_SHARED((2 * S,), jnp.float32),
        ],
    )
    def deg_kernel(idx_hbm, ones_hbm, zeros_hbm, out_hbm, idx_v, ones_v, acc):
        c = lax.axis_index("c")
        s = lax.axis_index("s")
        wid = c * NS + s
        pltpu.sync_copy(zeros_hbm.at[0, 0, pl.ds(s * ZCH, ZCH)],
                        acc.at[pl.ds(s * ZCH, ZCH)])
        pltpu.sync_copy(idx_hbm.at[wid], idx_v)
        pltpu.sync_copy(ones_hbm, ones_v)
        plsc.subcore_barrier()

        def body(j, carry):
            pltpu.sync_copy(ones_v, acc.at[idx_v.at[j]], add=True)
            return carry

        lax.fori_loop(0, KD, body, 0)
        plsc.subcore_barrier()
        pltpu.sync_copy(acc.at[pl.ds(s * ZCH, ZCH)],
                        out_hbm.at[c, 0, pl.ds(s * ZCH, ZCH)])

    return deg_kernel


def _make_edge_kernel(d):
    @functools.partial(
        pl.kernel,
        out_type=jax.ShapeDtypeStruct((NC, NP_, d), jnp.float32),
        mesh=_mesh,
        scratch_types=[
            pltpu.VMEM((K, C), jnp.int32),
            pltpu.VMEM((K, C), jnp.int32),
            pltpu.VMEM((C, d), jnp.float32),
            pltpu.VMEM_SHARED((NP_, d), jnp.float32),
            pltpu.SemaphoreType.DMA,
        ],
    )
    def edge_kernel(src_hbm, dst_hbm, ms_hbm, zeros_hbm, out_hbm,
                    src_v, dst_v, rows_v, acc, sem):
        c = lax.axis_index("c")
        s = lax.axis_index("s")
        wid = c * NS + s
        pltpu.sync_copy(zeros_hbm.at[pl.ds(s * RP, RP)],
                        acc.at[pl.ds(s * RP, RP)])
        pltpu.sync_copy(src_hbm.at[wid], src_v)
        pltpu.sync_copy(dst_hbm.at[wid], dst_v)
        plsc.subcore_barrier()

        def body(j, carry):
            pltpu.async_copy(ms_hbm.at[src_v.at[j]], rows_v, sem).wait()
            pltpu.sync_copy(rows_v, acc.at[dst_v.at[j]], add=True)
            return carry

        lax.fori_loop(0, K, body, 0)
        plsc.subcore_barrier()
        pltpu.sync_copy(acc.at[pl.ds(s * RP, RP)],
                        out_hbm.at[c, pl.ds(s * RP, RP)])

    return edge_kernel


_deg_call = _make_deg_kernel()
_edge_call_f = _make_edge_kernel(DF)


def _tc_front_body(x_ref, degp_ref, w1_ref, ms1_ref, dinv1_ref, dinv2_ref):
    x = x_ref[...]
    mean = jnp.mean(x, axis=0, keepdims=True)
    var = jnp.mean((x - mean) * (x - mean), axis=0, keepdims=True)
    h = (x - mean) * lax.rsqrt(var + EPS)
    degp = degp_ref[...]
    deg = degp[0] + degp[1] + 1.0
    dinv = lax.rsqrt(deg)
    d1 = dinv[0, :N]
    d2 = dinv[1, :N]
    m = jnp.dot(h, w1_ref[...], preferred_element_type=jnp.float32)
    ms1_ref[...] = m * d1[:, None]
    dinv1_ref[...] = d1[:, None]
    dinv2_ref[...] = d2[:, None]


def _tc_front(x, degp, w1):
    return pl.pallas_call(
        _tc_front_body,
        out_shape=[
            jax.ShapeDtypeStruct((N, DF), jnp.float32),
            jax.ShapeDtypeStruct((N, 1), jnp.float32),
            jax.ShapeDtypeStruct((N, 1), jnp.float32),
        ],
    )(x, degp, w1)


def _tc_mid_body(p1_ref, ms1_ref, dinv1_ref, b1_ref, w2_ref, dinv2_ref,
                 ms2_ref):
    p = p1_ref[0] + p1_ref[1] + ms1_ref[...]
    h1 = jnp.maximum(p * dinv1_ref[...] + b1_ref[...][None, :], 0.0)
    m2 = jnp.dot(h1, w2_ref[...], preferred_element_type=jnp.float32)
    ms2_ref[...] = m2 * dinv2_ref[...]


def _tc_mid(p1, ms1, dinv1, b1, w2p, dinv2):
    return pl.pallas_call(
        _tc_mid_body,
        out_shape=jax.ShapeDtypeStruct((N, DF), jnp.float32),
    )(p1, ms1, dinv1, b1, w2p, dinv2)


def _tc_final_body(p2_ref, ms2_ref, dinv2_ref, b2_ref, out_ref):
    p = (p2_ref[0] + p2_ref[1] + ms2_ref[...]) * dinv2_ref[...]
    out_ref[...] = p[:, :DC] + b2_ref[...][None, :]


def _tc_final(p2, ms2, dinv2, b2):
    return pl.pallas_call(
        _tc_final_body,
        out_shape=jax.ShapeDtypeStruct((N, DC), jnp.float32),
    )(p2, ms2, dinv2, b2)


@jax.jit
def _run(nodeblocks, x, w1, b1, w2, b2):
    nb = nodeblocks.astype(jnp.int32)
    src1 = nb[0, 0].reshape(NW, K, C)
    dst1 = nb[0, 1].reshape(NW, K, C)
    src2 = nb[1, 0].reshape(NW, K, C)
    dst2 = nb[1, 1].reshape(NW, K, C)
    degidx = jnp.concatenate([nb[0, 1], nb[1, 1] + S]).reshape(NW, KD, C)

    zeros_deg = jnp.zeros((1, 1, 2 * S), jnp.float32)
    zeros_f = jnp.zeros((NP_, DF), jnp.float32)
    ones_c = jnp.ones((C,), jnp.float32)
    w2p = jnp.pad(w2, ((0, 0), (0, DF - DC)))

    degp = _deg_call(degidx, ones_c, zeros_deg)
    degp = degp.reshape(NC, 2, S)
    ms1, dinv1, dinv2 = _tc_front(x, degp, w1)
    p1 = _edge_call_f(src1, dst1, ms1, zeros_f)
    ms2 = _tc_mid(p1[:, :N, :], ms1, dinv1, b1, w2p, dinv2)
    p2 = _edge_call_f(src2, dst2, ms2, zeros_f)
    return _tc_final(p2[:, :N, :], ms2, dinv2, b2)


def kernel(nodeblocks, x, W1, b1, W2, b2):
    return _run(nodeblocks, x, W1, b1, W2, b2)


# speedup vs baseline: 27.2716x; 1.1936x over previous
import functools

import jax
import jax.numpy as jnp
from jax import lax
from jax.experimental import pallas as pl
from jax.experimental.pallas import tpu as pltpu
from jax.experimental.pallas import tpu_sc as plsc

N = 10000
E = 320000
DF = 128
DC = 40
EPS = 1e-5

NC, NS = 2, 16
NW = NC * NS
EW = E // NW
C = 80
K = EW // C
KB = 25
NB = K // KB
KD = 2 * EW // C
S = 10240
ZCH = 2 * S // NS
NP_ = 10112
RP = NP_ // NS

_mesh = plsc.VectorSubcoreMesh(core_axis_name="c", subcore_axis_name="s")


def _make_deg_kernel():
    @functools.partial(
        pl.kernel,
        out_type=jax.ShapeDtypeStruct((NC, 1, 2 * S), jnp.float32),
        mesh=_mesh,
        scratch_types=[
            pltpu.VMEM((KD, C), jnp.int32),
            pltpu.VMEM((C,), jnp.float32),
            pltpu.VMEM_SHARED((2 * S,), jnp.float32),
        ],
    )
    def deg_kernel(idx_hbm, ones_hbm, zeros_hbm, out_hbm, idx_v, ones_v, acc):
        c = lax.axis_index("c")
        s = lax.axis_index("s")
        wid = c * NS + s
        pltpu.sync_copy(zeros_hbm.at[0, 0, pl.ds(s * ZCH, ZCH)],
                        acc.at[pl.ds(s * ZCH, ZCH)])
        pltpu.sync_copy(idx_hbm.at[wid], idx_v)
        pltpu.sync_copy(ones_hbm, ones_v)
        plsc.subcore_barrier()

        def body(j, carry):
            pltpu.sync_copy(ones_v, acc.at[idx_v.at[j]], add=True)
            return carry

        lax.fori_loop(0, KD, body, 0)
        plsc.subcore_barrier()
        pltpu.sync_copy(acc.at[pl.ds(s * ZCH, ZCH)],
                        out_hbm.at[c, 0, pl.ds(s * ZCH, ZCH)])

    return deg_kernel


def _make_edge_kernel(d):
    @functools.partial(
        pl.kernel,
        out_type=jax.ShapeDtypeStruct((NC, NP_, d), jnp.float32),
        mesh=_mesh,
        scratch_types=[
            pltpu.VMEM((KB, C), jnp.int32),
            pltpu.VMEM((KB, C), jnp.int32),
            pltpu.VMEM((C, d), jnp.float32),
            pltpu.VMEM((C, d), jnp.float32),
            pltpu.VMEM_SHARED((NP_, d), jnp.float32),
            pltpu.SemaphoreType.DMA,
        ],
    )
    def edge_kernel(src_hbm, dst_hbm, ms_hbm, zeros_hbm, out_hbm,
                    src_v, dst_v, rows0_v, rows1_v, acc, sem):
        c = lax.axis_index("c")
        s = lax.axis_index("s")
        wid = c * NS + s
        pltpu.sync_copy(zeros_hbm.at[pl.ds(s * RP, RP)],
                        acc.at[pl.ds(s * RP, RP)])
        plsc.subcore_barrier()

        def step(cur, nxt, j):
            pltpu.make_async_copy(ms_hbm.at[src_v.at[j]], cur, sem).wait()

            @pl.when(j + 1 < KB)
            def _():
                pltpu.async_copy(ms_hbm.at[src_v.at[j + 1]], nxt, sem)

            pltpu.sync_copy(cur, acc.at[dst_v.at[j]], add=True)

        def inner(j, carry):
            @pl.when(j % 2 == 0)
            def _():
                step(rows0_v, rows1_v, j)

            @pl.when(j % 2 == 1)
            def _():
                step(rows1_v, rows0_v, j)

            return carry

        def block(b, carry):
            pltpu.sync_copy(src_hbm.at[wid, b], src_v)
            pltpu.sync_copy(dst_hbm.at[wid, b], dst_v)
            pltpu.async_copy(ms_hbm.at[src_v.at[0]], rows0_v, sem)
            lax.fori_loop(0, KB, inner, 0)
            return carry

        lax.fori_loop(0, NB, block, 0)
        plsc.subcore_barrier()
        pltpu.sync_copy(acc.at[pl.ds(s * RP, RP)],
                        out_hbm.at[c, pl.ds(s * RP, RP)])

    return edge_kernel


_deg_call = _make_deg_kernel()
_edge_call_f = _make_edge_kernel(DF)


def _tc_front_body(x_ref, degp_ref, w1_ref, ms1_ref, dinv1_ref, dinv2_ref):
    x = x_ref[...]
    mean = jnp.mean(x, axis=0, keepdims=True)
    var = jnp.mean((x - mean) * (x - mean), axis=0, keepdims=True)
    h = (x - mean) * lax.rsqrt(var + EPS)
    degp = degp_ref[...]
    deg = degp[0] + degp[1] + 1.0
    dinv = lax.rsqrt(deg)
    d1 = dinv[0, :N]
    d2 = dinv[1, :N]
    m = jnp.dot(h, w1_ref[...], preferred_element_type=jnp.float32)
    ms1_ref[...] = m * d1[:, None]
    dinv1_ref[...] = d1[:, None]
    dinv2_ref[...] = d2[:, None]


def _tc_front(x, degp, w1):
    return pl.pallas_call(
        _tc_front_body,
        out_shape=[
            jax.ShapeDtypeStruct((N, DF), jnp.float32),
            jax.ShapeDtypeStruct((N, 1), jnp.float32),
            jax.ShapeDtypeStruct((N, 1), jnp.float32),
        ],
    )(x, degp, w1)


def _tc_mid_body(p1_ref, ms1_ref, dinv1_ref, b1_ref, w2_ref, dinv2_ref,
                 ms2_ref):
    p = p1_ref[0] + p1_ref[1] + ms1_ref[...]
    h1 = jnp.maximum(p * dinv1_ref[...] + b1_ref[...][None, :], 0.0)
    m2 = jnp.dot(h1, w2_ref[...], preferred_element_type=jnp.float32)
    ms2_ref[...] = m2 * dinv2_ref[...]


def _tc_mid(p1, ms1, dinv1, b1, w2p, dinv2):
    return pl.pallas_call(
        _tc_mid_body,
        out_shape=jax.ShapeDtypeStruct((N, DF), jnp.float32),
    )(p1, ms1, dinv1, b1, w2p, dinv2)


def _tc_final_body(p2_ref, ms2_ref, dinv2_ref, b2_ref, out_ref):
    p = (p2_ref[0] + p2_ref[1] + ms2_ref[...]) * dinv2_ref[...]
    out_ref[...] = p[:, :DC] + b2_ref[...][None, :]


def _tc_final(p2, ms2, dinv2, b2):
    return pl.pallas_call(
        _tc_final_body,
        out_shape=jax.ShapeDtypeStruct((N, DC), jnp.float32),
    )(p2, ms2, dinv2, b2)


@jax.jit
def _run(nodeblocks, x, w1, b1, w2, b2):
    nb = nodeblocks.astype(jnp.int32)
    src1 = nb[0, 0].reshape(NW, NB, KB, C)
    dst1 = nb[0, 1].reshape(NW, NB, KB, C)
    src2 = nb[1, 0].reshape(NW, NB, KB, C)
    dst2 = nb[1, 1].reshape(NW, NB, KB, C)
    degidx = jnp.concatenate([nb[0, 1], nb[1, 1] + S]).reshape(NW, KD, C)

    zeros_deg = jnp.zeros((1, 1, 2 * S), jnp.float32)
    zeros_f = jnp.zeros((NP_, DF), jnp.float32)
    ones_c = jnp.ones((C,), jnp.float32)
    w2p = jnp.pad(w2, ((0, 0), (0, DF - DC)))

    degp = _deg_call(degidx, ones_c, zeros_deg)
    degp = degp.reshape(NC, 2, S)
    ms1, dinv1, dinv2 = _tc_front(x, degp, w1)
    p1 = _edge_call_f(src1, dst1, ms1, zeros_f)
    ms2 = _tc_mid(p1[:, :N, :], ms1, dinv1, b1, w2p, dinv2)
    p2 = _edge_call_f(src2, dst2, ms2, zeros_f)
    return _tc_final(p2[:, :N, :], ms2, dinv2, b2)


def kernel(nodeblocks, x, W1, b1, W2, b2):
    return _run(nodeblocks, x, W1, b1, W2, b2)


# speedup vs baseline: 29.4922x; 1.0814x over previous
import functools

import jax
import jax.numpy as jnp
from jax import lax
from jax.experimental import pallas as pl
from jax.experimental.pallas import tpu as pltpu
from jax.experimental.pallas import tpu_sc as plsc

N = 10000
E = 320000
DF = 128
DC = 40
EPS = 1e-5

NC, NS = 2, 16
NW = NC * NS
EW = E // NW
C = 80
K = EW // C
KB = 25
NB = K // KB
KD = 2 * EW // C
S = 10240
ZCH = 2 * S // NS
NP_ = 10112
RP = NP_ // NS

_mesh = plsc.VectorSubcoreMesh(core_axis_name="c", subcore_axis_name="s")


def _make_deg_kernel():
    @functools.partial(
        pl.kernel,
        out_type=jax.ShapeDtypeStruct((NC, 1, 2 * S), jnp.float32),
        mesh=_mesh,
        scratch_types=[
            pltpu.VMEM((KD, C), jnp.int32),
            pltpu.VMEM((C,), jnp.float32),
            pltpu.VMEM_SHARED((2 * S,), jnp.float32),
        ],
    )
    def deg_kernel(idx_hbm, ones_hbm, zeros_hbm, out_hbm, idx_v, ones_v, acc):
        c = lax.axis_index("c")
        s = lax.axis_index("s")
        wid = c * NS + s
        pltpu.sync_copy(zeros_hbm.at[0, 0, pl.ds(s * ZCH, ZCH)],
                        acc.at[pl.ds(s * ZCH, ZCH)])
        pltpu.sync_copy(idx_hbm.at[wid], idx_v)
        pltpu.sync_copy(ones_hbm, ones_v)
        plsc.subcore_barrier()

        def body(j, carry):
            pltpu.sync_copy(ones_v, acc.at[idx_v.at[j]], add=True)
            return carry

        lax.fori_loop(0, KD, body, 0)
        plsc.subcore_barrier()
        pltpu.sync_copy(acc.at[pl.ds(s * ZCH, ZCH)],
                        out_hbm.at[c, 0, pl.ds(s * ZCH, ZCH)])

    return deg_kernel


def _make_edge_kernel(d, tc_tiling=True):
    @functools.partial(
        pl.kernel,
        out_type=jax.ShapeDtypeStruct((NC, NP_, d), jnp.float32),
        mesh=_mesh,
        compiler_params=pltpu.CompilerParams(use_tc_tiling_on_sc=tc_tiling),
        scratch_types=[
            pltpu.VMEM((KB, C), jnp.int32),
            pltpu.VMEM((KB, C), jnp.int32),
            pltpu.VMEM((C, d), jnp.float32),
            pltpu.VMEM((C, d), jnp.float32),
            pltpu.VMEM_SHARED((NP_, d), jnp.float32),
            pltpu.SemaphoreType.DMA,
        ],
    )
    def edge_kernel(src_hbm, dst_hbm, ms_hbm, zeros_hbm, out_hbm,
                    src_v, dst_v, rows0_v, rows1_v, acc, sem):
        c = lax.axis_index("c")
        s = lax.axis_index("s")
        wid = c * NS + s
        pltpu.sync_copy(zeros_hbm.at[pl.ds(s * RP, RP)],
                        acc.at[pl.ds(s * RP, RP)])
        plsc.subcore_barrier()

        def step(cur, nxt, j):
            pltpu.make_async_copy(ms_hbm.at[src_v.at[j]], cur, sem).wait()

            @pl.when(j + 1 < KB)
            def _():
                pltpu.async_copy(ms_hbm.at[src_v.at[j + 1]], nxt, sem)

            pltpu.sync_copy(cur, acc.at[dst_v.at[j]], add=True)

        def inner(j, carry):
            @pl.when(j % 2 == 0)
            def _():
                step(rows0_v, rows1_v, j)

            @pl.when(j % 2 == 1)
            def _():
                step(rows1_v, rows0_v, j)

            return carry

        def block(b, carry):
            pltpu.sync_copy(src_hbm.at[wid, b], src_v)
            pltpu.sync_copy(dst_hbm.at[wid, b], dst_v)
            pltpu.async_copy(ms_hbm.at[src_v.at[0]], rows0_v, sem)
            lax.fori_loop(0, KB, inner, 0)
            return carry

        lax.fori_loop(0, NB, block, 0)
        plsc.subcore_barrier()
        pltpu.sync_copy(acc.at[pl.ds(s * RP, RP)],
                        out_hbm.at[c, pl.ds(s * RP, RP)])

    return edge_kernel


_deg_call = _make_deg_kernel()
_edge_call_f = _make_edge_kernel(DF)
_edge_call_c = _make_edge_kernel(DC, tc_tiling=False)


def _tc_front_body(x_ref, degp_ref, w1_ref, ms1_ref, dinv1_ref, dinv2_ref):
    x = x_ref[...]
    mean = jnp.mean(x, axis=0, keepdims=True)
    var = jnp.mean((x - mean) * (x - mean), axis=0, keepdims=True)
    h = (x - mean) * lax.rsqrt(var + EPS)
    degp = degp_ref[...]
    deg = degp[0] + degp[1] + 1.0
    dinv = lax.rsqrt(deg)
    d1 = dinv[0, :N]
    d2 = dinv[1, :N]
    m = jnp.dot(h, w1_ref[...], preferred_element_type=jnp.float32)
    ms1_ref[...] = m * d1[:, None]
    dinv1_ref[...] = d1[:, None]
    dinv2_ref[...] = d2[:, None]


def _tc_front(x, degp, w1):
    return pl.pallas_call(
        _tc_front_body,
        out_shape=[
            jax.ShapeDtypeStruct((N, DF), jnp.float32),
            jax.ShapeDtypeStruct((N, 1), jnp.float32),
            jax.ShapeDtypeStruct((N, 1), jnp.float32),
        ],
    )(x, degp, w1)


def _tc_mid_body(p1_ref, ms1_ref, dinv1_ref, b1_ref, w2_ref, dinv2_ref,
                 ms2_ref):
    p = p1_ref[0] + p1_ref[1] + ms1_ref[...]
    h1 = jnp.maximum(p * dinv1_ref[...] + b1_ref[...][None, :], 0.0)
    m2 = jnp.dot(h1, w2_ref[...], preferred_element_type=jnp.float32)
    ms2_ref[...] = m2 * dinv2_ref[...]


def _tc_mid(p1, ms1, dinv1, b1, w2, dinv2):
    return pl.pallas_call(
        _tc_mid_body,
        out_shape=jax.ShapeDtypeStruct((N, DC), jnp.float32),
    )(p1, ms1, dinv1, b1, w2, dinv2)


def _tc_final_body(p2_ref, ms2_ref, dinv2_ref, b2_ref, out_ref):
    p = (p2_ref[0] + p2_ref[1] + ms2_ref[...]) * dinv2_ref[...]
    out_ref[...] = p + b2_ref[...][None, :]


def _tc_final(p2, ms2, dinv2, b2):
    return pl.pallas_call(
        _tc_final_body,
        out_shape=jax.ShapeDtypeStruct((N, DC), jnp.float32),
    )(p2, ms2, dinv2, b2)


@jax.jit
def _run(nodeblocks, x, w1, b1, w2, b2):
    nb = nodeblocks.astype(jnp.int32)
    src1 = nb[0, 0].reshape(NW, NB, KB, C)
    dst1 = nb[0, 1].reshape(NW, NB, KB, C)
    src2 = nb[1, 0].reshape(NW, NB, KB, C)
    dst2 = nb[1, 1].reshape(NW, NB, KB, C)
    degidx = jnp.concatenate([nb[0, 1], nb[1, 1] + S]).reshape(NW, KD, C)

    zeros_deg = jnp.zeros((1, 1, 2 * S), jnp.float32)
    zeros_f = jnp.zeros((NP_, DF), jnp.float32)
    zeros_c = jnp.zeros((NP_, DC), jnp.float32)
    ones_c = jnp.ones((C,), jnp.float32)

    degp = _deg_call(degidx, ones_c, zeros_deg)
    degp = degp.reshape(NC, 2, S)
    ms1, dinv1, dinv2 = _tc_front(x, degp, w1)
    p1 = _edge_call_f(src1, dst1, ms1, zeros_f)
    ms2 = _tc_mid(p1[:, :N, :], ms1, dinv1, b1, w2, dinv2)
    p2 = _edge_call_c(src2, dst2, ms2, zeros_c)
    return _tc_final(p2[:, :N, :], ms2, dinv2, b2)


def kernel(nodeblocks, x, W1, b1, W2, b2):
    return _run(nodeblocks, x, W1, b1, W2, b2)


# speedup vs baseline: 30.4347x; 1.0320x over previous
import functools

import jax
import jax.numpy as jnp
from jax import lax
from jax.experimental import pallas as pl
from jax.experimental.pallas import tpu as pltpu
from jax.experimental.pallas import tpu_sc as plsc

N = 10000
E = 320000
DF = 128
DC = 40
EPS = 1e-5

NC, NS = 2, 16
NW = NC * NS
EW = E // NW
C = 80
K = EW // C
KB = 25
NB = K // KB
KD = 2 * EW // C
S = 10240
ZCH = 2 * S // NS
NP_ = 10112
RP = NP_ // NS

_mesh = plsc.VectorSubcoreMesh(core_axis_name="c", subcore_axis_name="s")


def _make_deg_kernel():
    @functools.partial(
        pl.kernel,
        out_type=jax.ShapeDtypeStruct((NC, 1, 2 * S), jnp.float32),
        mesh=_mesh,
        scratch_types=[
            pltpu.VMEM((KD, C), jnp.int32),
            pltpu.VMEM((C,), jnp.float32),
            pltpu.VMEM_SHARED((2 * S,), jnp.float32),
            pltpu.SemaphoreType.DMA,
        ],
    )
    def deg_kernel(idx_hbm, ones_hbm, zeros_hbm, out_hbm, idx_v, ones_v, acc,
                   sem):
        c = lax.axis_index("c")
        s = lax.axis_index("s")
        wid = c * NS + s
        pltpu.sync_copy(zeros_hbm.at[0, 0, pl.ds(s * ZCH, ZCH)],
                        acc.at[pl.ds(s * ZCH, ZCH)])
        pltpu.sync_copy(idx_hbm.at[wid], idx_v)
        pltpu.sync_copy(ones_hbm, ones_v)
        plsc.subcore_barrier()

        def body(j, carry):
            pltpu.async_copy(ones_v, acc.at[idx_v.at[j]], sem, add=True)
            return carry

        lax.fori_loop(0, KD, body, 0)

        def drain(j, carry):
            pltpu.make_async_copy(ones_v, acc.at[idx_v.at[0]], sem).wait()
            return carry

        lax.fori_loop(0, KD, drain, 0)
        plsc.subcore_barrier()
        pltpu.sync_copy(acc.at[pl.ds(s * ZCH, ZCH)],
                        out_hbm.at[c, 0, pl.ds(s * ZCH, ZCH)])

    return deg_kernel


def _make_edge_kernel(d, tc_tiling=True):
    @functools.partial(
        pl.kernel,
        out_type=jax.ShapeDtypeStruct((NC, NP_, d), jnp.float32),
        mesh=_mesh,
        compiler_params=pltpu.CompilerParams(use_tc_tiling_on_sc=tc_tiling),
        scratch_types=[
            pltpu.VMEM((KB, C), jnp.int32),
            pltpu.VMEM((KB, C), jnp.int32),
            pltpu.VMEM((C, d), jnp.float32),
            pltpu.VMEM((C, d), jnp.float32),
            pltpu.VMEM((C, d), jnp.float32),
            pltpu.VMEM_SHARED((NP_, d), jnp.float32),
            pltpu.SemaphoreType.DMA,
            pltpu.SemaphoreType.DMA,
        ],
    )
    def edge_kernel(src_hbm, dst_hbm, ms_hbm, zeros_hbm, out_hbm,
                    src_v, dst_v, rows0_v, rows1_v, rows2_v, acc,
                    gsem, ssem):
        c = lax.axis_index("c")
        s = lax.axis_index("s")
        wid = c * NS + s
        bufs = (rows0_v, rows1_v, rows2_v)
        pltpu.sync_copy(zeros_hbm.at[pl.ds(s * RP, RP)],
                        acc.at[pl.ds(s * RP, RP)])
        plsc.subcore_barrier()

        def step(cur, nxt, j):
            pltpu.make_async_copy(ms_hbm.at[src_v.at[j]], cur, gsem).wait()

            @pl.when(j >= 2)
            def _():
                pltpu.make_async_copy(nxt, acc.at[dst_v.at[0]], ssem).wait()

            @pl.when(j + 1 < KB)
            def _():
                pltpu.async_copy(ms_hbm.at[src_v.at[j + 1]], nxt, gsem)

            pltpu.async_copy(cur, acc.at[dst_v.at[j]], ssem, add=True)

        def inner(j, carry):
            for r in range(3):
                @pl.when(j % 3 == r)
                def _(r=r):
                    step(bufs[r], bufs[(r + 1) % 3], j)

            return carry

        def block(b, carry):
            pltpu.sync_copy(src_hbm.at[wid, b], src_v)
            pltpu.sync_copy(dst_hbm.at[wid, b], dst_v)
            pltpu.async_copy(ms_hbm.at[src_v.at[0]], rows0_v, gsem)
            lax.fori_loop(0, KB, inner, 0)

            def drain(j, carry2):
                pltpu.make_async_copy(rows0_v, acc.at[dst_v.at[0]],
                                      ssem).wait()
                return carry2

            lax.fori_loop(0, 2, drain, 0)
            return carry

        lax.fori_loop(0, NB, block, 0)
        plsc.subcore_barrier()
        pltpu.sync_copy(acc.at[pl.ds(s * RP, RP)],
                        out_hbm.at[c, pl.ds(s * RP, RP)])

    return edge_kernel


_deg_call = _make_deg_kernel()
_edge_call_f = _make_edge_kernel(DF)
_edge_call_c = _make_edge_kernel(DC, tc_tiling=False)


def _tc_front_body(x_ref, degp_ref, w1_ref, ms1_ref, dinv1_ref, dinv2_ref):
    x = x_ref[...]
    mean = jnp.mean(x, axis=0, keepdims=True)
    var = jnp.mean((x - mean) * (x - mean), axis=0, keepdims=True)
    h = (x - mean) * lax.rsqrt(var + EPS)
    degp = degp_ref[...]
    deg = degp[0] + degp[1] + 1.0
    dinv = lax.rsqrt(deg)
    d1 = dinv[0, :N]
    d2 = dinv[1, :N]
    m = jnp.dot(h, w1_ref[...], preferred_element_type=jnp.float32)
    ms1_ref[...] = m * d1[:, None]
    dinv1_ref[...] = d1[:, None]
    dinv2_ref[...] = d2[:, None]


def _tc_front(x, degp, w1):
    return pl.pallas_call(
        _tc_front_body,
        out_shape=[
            jax.ShapeDtypeStruct((N, DF), jnp.float32),
            jax.ShapeDtypeStruct((N, 1), jnp.float32),
            jax.ShapeDtypeStruct((N, 1), jnp.float32),
        ],
    )(x, degp, w1)


def _tc_mid_body(p1_ref, ms1_ref, dinv1_ref, b1_ref, w2_ref, dinv2_ref,
                 ms2_ref):
    p = p1_ref[0] + p1_ref[1] + ms1_ref[...]
    h1 = jnp.maximum(p * dinv1_ref[...] + b1_ref[...][None, :], 0.0)
    m2 = jnp.dot(h1, w2_ref[...], preferred_element_type=jnp.float32)
    ms2_ref[...] = m2 * dinv2_ref[...]


def _tc_mid(p1, ms1, dinv1, b1, w2, dinv2):
    return pl.pallas_call(
        _tc_mid_body,
        out_shape=jax.ShapeDtypeStruct((N, DC), jnp.float32),
    )(p1, ms1, dinv1, b1, w2, dinv2)


def _tc_final_body(p2_ref, ms2_ref, dinv2_ref, b2_ref, out_ref):
    p = (p2_ref[0] + p2_ref[1] + ms2_ref[...]) * dinv2_ref[...]
    out_ref[...] = p + b2_ref[...][None, :]


def _tc_final(p2, ms2, dinv2, b2):
    return pl.pallas_call(
        _tc_final_body,
        out_shape=jax.ShapeDtypeStruct((N, DC), jnp.float32),
    )(p2, ms2, dinv2, b2)


@jax.jit
def _run(nodeblocks, x, w1, b1, w2, b2):
    nb = nodeblocks.astype(jnp.int32)
    src1 = nb[0, 0].reshape(NW, NB, KB, C)
    dst1 = nb[0, 1].reshape(NW, NB, KB, C)
    src2 = nb[1, 0].reshape(NW, NB, KB, C)
    dst2 = nb[1, 1].reshape(NW, NB, KB, C)
    degidx = jnp.concatenate([nb[0, 1], nb[1, 1] + S]).reshape(NW, KD, C)

    zeros_deg = jnp.zeros((1, 1, 2 * S), jnp.float32)
    zeros_f = jnp.zeros((NP_, DF), jnp.float32)
    zeros_c = jnp.zeros((NP_, DC), jnp.float32)
    ones_c = jnp.ones((C,), jnp.float32)

    degp = _deg_call(degidx, ones_c, zeros_deg)
    degp = degp.reshape(NC, 2, S)
    ms1, dinv1, dinv2 = _tc_front(x, degp, w1)
    p1 = _edge_call_f(src1, dst1, ms1, zeros_f)
    ms2 = _tc_mid(p1[:, :N, :], ms1, dinv1, b1, w2, dinv2)
    p2 = _edge_call_c(src2, dst2, ms2, zeros_c)
    return _tc_final(p2[:, :N, :], ms2, dinv2, b2)


def kernel(nodeblocks, x, W1, b1, W2, b2):
    return _run(nodeblocks, x, W1, b1, W2, b2)


# speedup vs baseline: 31.1918x; 1.0249x over previous
import functools

import jax
import jax.numpy as jnp
from jax import lax
from jax.experimental import pallas as pl
from jax.experimental.pallas import tpu as pltpu
from jax.experimental.pallas import tpu_sc as plsc

N = 10000
E = 320000
DF = 128
DC = 40
EPS = 1e-5

NC, NS = 2, 16
NW = NC * NS
EW = E // NW
C = 80
K = EW // C
KB = 25
NB = K // KB
S = 10240
SCH = S // NS
NP_ = 10112
RP = NP_ // NS

_mesh = plsc.VectorSubcoreMesh(core_axis_name="c", subcore_axis_name="s")


def _make_deg_kernel():
    @functools.partial(
        pl.kernel,
        out_type=jax.ShapeDtypeStruct((NC, 2, 1, S), jnp.float32),
        mesh=_mesh,
        scratch_types=[
            pltpu.VMEM((KB, C), jnp.int32),
            pltpu.VMEM((KB, C), jnp.int32),
            pltpu.VMEM((C,), jnp.float32),
            pltpu.VMEM_SHARED((S,), jnp.float32),
            pltpu.VMEM_SHARED((S,), jnp.float32),
            pltpu.SemaphoreType.DMA,
        ],
    )
    def deg_kernel(nb_hbm, ones_hbm, zeros_hbm, out_hbm,
                   idx0_v, idx1_v, ones_v, acc0, acc1, sem):
        c = lax.axis_index("c")
        s = lax.axis_index("s")
        wid = c * NS + s
        idx_bufs = (idx0_v, idx1_v)
        accs = (acc0, acc1)
        pltpu.sync_copy(zeros_hbm.at[0, 0, pl.ds(s * SCH, SCH)],
                        acc0.at[pl.ds(s * SCH, SCH)])
        pltpu.sync_copy(zeros_hbm.at[0, 0, pl.ds(s * SCH, SCH)],
                        acc1.at[pl.ds(s * SCH, SCH)])
        pltpu.sync_copy(ones_hbm, ones_v)
        plsc.subcore_barrier()

        for l in range(2):
            for b in range(NB):
                k = l * NB + b
                buf = idx_bufs[k % 2]
                acc = accs[l]
                if k >= 2:
                    def drain(j, carry):
                        pltpu.make_async_copy(
                            ones_v, acc0.at[idx0_v.at[0]], sem).wait()
                        return carry

                    lax.fori_loop(0, KB, drain, 0)
                pltpu.sync_copy(nb_hbm.at[l, 1, wid, b], buf)

                def body(j, carry, buf=buf, acc=acc):
                    pltpu.async_copy(ones_v, acc.at[buf.at[j]], sem,
                                     add=True)
                    return carry

                lax.fori_loop(0, KB, body, 0)

        def drain_tail(j, carry):
            pltpu.make_async_copy(ones_v, acc0.at[idx0_v.at[0]], sem).wait()
            return carry

        lax.fori_loop(0, 2 * KB, drain_tail, 0)
        plsc.subcore_barrier()
        pltpu.sync_copy(acc0.at[pl.ds(s * SCH, SCH)],
                        out_hbm.at[c, 0, 0, pl.ds(s * SCH, SCH)])
        pltpu.sync_copy(acc1.at[pl.ds(s * SCH, SCH)],
                        out_hbm.at[c, 1, 0, pl.ds(s * SCH, SCH)])

    return deg_kernel


def _make_edge_kernel(d, layer, tc_tiling=True):
    @functools.partial(
        pl.kernel,
        out_type=jax.ShapeDtypeStruct((NC, NP_, d), jnp.float32),
        mesh=_mesh,
        compiler_params=pltpu.CompilerParams(use_tc_tiling_on_sc=tc_tiling),
        scratch_types=[
            pltpu.VMEM((KB, C), jnp.int32),
            pltpu.VMEM((KB, C), jnp.int32),
            pltpu.VMEM((C, d), jnp.float32),
            pltpu.VMEM((C, d), jnp.float32),
            pltpu.VMEM((C, d), jnp.float32),
            pltpu.VMEM_SHARED((NP_, d), jnp.float32),
            pltpu.SemaphoreType.DMA,
            pltpu.SemaphoreType.DMA,
        ],
    )
    def edge_kernel(nb_hbm, ms_hbm, zeros_hbm, out_hbm,
                    src_v, dst_v, rows0_v, rows1_v, rows2_v, acc,
                    gsem, ssem):
        c = lax.axis_index("c")
        s = lax.axis_index("s")
        wid = c * NS + s
        bufs = (rows0_v, rows1_v, rows2_v)
        pltpu.sync_copy(zeros_hbm.at[pl.ds(s * RP, RP)],
                        acc.at[pl.ds(s * RP, RP)])
        plsc.subcore_barrier()

        def step(cur, nxt, j):
            pltpu.make_async_copy(ms_hbm.at[src_v.at[j]], cur, gsem).wait()

            @pl.when(j >= 2)
            def _():
                pltpu.make_async_copy(nxt, acc.at[dst_v.at[0]], ssem).wait()

            @pl.when(j + 1 < KB)
            def _():
                pltpu.async_copy(ms_hbm.at[src_v.at[j + 1]], nxt, gsem)

            pltpu.async_copy(cur, acc.at[dst_v.at[j]], ssem, add=True)

        def inner(j, carry):
            for r in range(3):
                @pl.when(j % 3 == r)
                def _(r=r):
                    step(bufs[r], bufs[(r + 1) % 3], j)

            return carry

        def block(b, carry):
            pltpu.sync_copy(nb_hbm.at[layer, 0, wid, b], src_v)
            pltpu.sync_copy(nb_hbm.at[layer, 1, wid, b], dst_v)
            pltpu.async_copy(ms_hbm.at[src_v.at[0]], rows0_v, gsem)
            lax.fori_loop(0, KB, inner, 0)

            def drain(j, carry2):
                pltpu.make_async_copy(rows0_v, acc.at[dst_v.at[0]],
                                      ssem).wait()
                return carry2

            lax.fori_loop(0, 2, drain, 0)
            return carry

        lax.fori_loop(0, NB, block, 0)
        plsc.subcore_barrier()
        pltpu.sync_copy(acc.at[pl.ds(s * RP, RP)],
                        out_hbm.at[c, pl.ds(s * RP, RP)])

    return edge_kernel


_deg_call = _make_deg_kernel()
_edge_call_1 = _make_edge_kernel(DF, 0)
_edge_call_2 = _make_edge_kernel(DC, 1, tc_tiling=False)


def _tc_mm_body(x_ref, w1_ref, m1_ref):
    x = x_ref[...]
    mean = jnp.mean(x, axis=0, keepdims=True)
    var = jnp.mean((x - mean) * (x - mean), axis=0, keepdims=True)
    h = (x - mean) * lax.rsqrt(var + EPS)
    m1_ref[...] = jnp.dot(h, w1_ref[...], preferred_element_type=jnp.float32)


def _tc_mm(x, w1):
    return pl.pallas_call(
        _tc_mm_body,
        out_shape=jax.ShapeDtypeStruct((N, DF), jnp.float32),
    )(x, w1)


def _tc_scale_body(m1_ref, degp_ref, ms1_ref, dinv1_ref, dinv2_ref):
    degp = degp_ref[...]
    deg = degp[0] + degp[1] + 1.0
    dinv = lax.rsqrt(deg)
    d1 = dinv[0, :N]
    d2 = dinv[1, :N]
    ms1_ref[...] = m1_ref[...] * d1[:, None]
    dinv1_ref[...] = d1[:, None]
    dinv2_ref[...] = d2[:, None]


def _tc_scale(m1, degp):
    return pl.pallas_call(
        _tc_scale_body,
        out_shape=[
            jax.ShapeDtypeStruct((N, DF), jnp.float32),
            jax.ShapeDtypeStruct((N, 1), jnp.float32),
            jax.ShapeDtypeStruct((N, 1), jnp.float32),
        ],
    )(m1, degp)


def _tc_mid_body(p1_ref, ms1_ref, dinv1_ref, b1_ref, w2_ref, dinv2_ref,
                 ms2_ref):
    p = p1_ref[0] + p1_ref[1] + ms1_ref[...]
    h1 = jnp.maximum(p * dinv1_ref[...] + b1_ref[...][None, :], 0.0)
    m2 = jnp.dot(h1, w2_ref[...], preferred_element_type=jnp.float32)
    ms2_ref[...] = m2 * dinv2_ref[...]


def _tc_mid(p1, ms1, dinv1, b1, w2, dinv2):
    return pl.pallas_call(
        _tc_mid_body,
        out_shape=jax.ShapeDtypeStruct((N, DC), jnp.float32),
    )(p1, ms1, dinv1, b1, w2, dinv2)


def _tc_final_body(p2_ref, ms2_ref, dinv2_ref, b2_ref, out_ref):
    p = (p2_ref[0] + p2_ref[1] + ms2_ref[...]) * dinv2_ref[...]
    out_ref[...] = p + b2_ref[...][None, :]


def _tc_final(p2, ms2, dinv2, b2):
    return pl.pallas_call(
        _tc_final_body,
        out_shape=jax.ShapeDtypeStruct((N, DC), jnp.float32),
    )(p2, ms2, dinv2, b2)


@jax.jit
def _run(nodeblocks, x, w1, b1, w2, b2):
    nb6 = nodeblocks.astype(jnp.int32).reshape(2, 2, NW, NB, KB, C)

    zeros_deg = jnp.zeros((1, 1, S), jnp.float32)
    zeros_f = jnp.zeros((NP_, DF), jnp.float32)
    zeros_c = jnp.zeros((NP_, DC), jnp.float32)
    ones_c = jnp.ones((C,), jnp.float32)

    m1 = _tc_mm(x, w1)
    degp = _deg_call(nb6, ones_c, zeros_deg)
    ms1, dinv1, dinv2 = _tc_scale(m1, degp.reshape(NC, 2, S))
    p1 = _edge_call_1(nb6, ms1, zeros_f)
    ms2 = _tc_mid(p1[:, :N, :], ms1, dinv1, b1, w2, dinv2)
    p2 = _edge_call_2(nb6, ms2, zeros_c)
    return _tc_final(p2[:, :N, :], ms2, dinv2, b2)


def kernel(nodeblocks, x, W1, b1, W2, b2):
    return _run(nodeblocks, x, W1, b1, W2, b2)


# speedup vs baseline: 36.6734x; 1.1757x over previous
import functools

import jax
import jax.numpy as jnp
from jax import lax
from jax.experimental import pallas as pl
from jax.experimental.pallas import tpu as pltpu
from jax.experimental.pallas import tpu_sc as plsc

N = 10000
E = 320000
DF = 128
DC = 40
EPS = 1e-5

NC, NS = 2, 16
NW = NC * NS
EW = E // NW
C = 80
K = EW // C
KB = 25
NB = K // KB
S = 10240
SCH = S // NS
NP_ = 10112
RP = NP_ // NS

_mesh = plsc.VectorSubcoreMesh(core_axis_name="c", subcore_axis_name="s")


def _make_deg_kernel():
    @functools.partial(
        pl.kernel,
        out_type=jax.ShapeDtypeStruct((NC, 2, 1, S), jnp.float32),
        mesh=_mesh,
        scratch_types=[
            pltpu.VMEM((KB, C), jnp.int32),
            pltpu.VMEM((KB, C), jnp.int32),
            pltpu.VMEM((C,), jnp.float32),
            pltpu.VMEM_SHARED((S,), jnp.float32),
            pltpu.VMEM_SHARED((S,), jnp.float32),
            pltpu.SemaphoreType.DMA,
        ],
    )
    def deg_kernel(nb_hbm, ones_hbm, zeros_hbm, out_hbm,
                   idx0_v, idx1_v, ones_v, acc0, acc1, sem):
        c = lax.axis_index("c")
        s = lax.axis_index("s")
        wid = c * NS + s
        idx_bufs = (idx0_v, idx1_v)
        accs = (acc0, acc1)
        pltpu.sync_copy(zeros_hbm.at[0, 0, pl.ds(s * SCH, SCH)],
                        acc0.at[pl.ds(s * SCH, SCH)])
        pltpu.sync_copy(zeros_hbm.at[0, 0, pl.ds(s * SCH, SCH)],
                        acc1.at[pl.ds(s * SCH, SCH)])
        pltpu.sync_copy(ones_hbm, ones_v)
        plsc.subcore_barrier()

        for l in range(2):
            for b in range(NB):
                k = l * NB + b
                buf = idx_bufs[k % 2]
                acc = accs[l]
                if k >= 2:
                    def drain(j, carry):
                        pltpu.make_async_copy(
                            ones_v, acc0.at[idx0_v.at[0]], sem).wait()
                        return carry

                    lax.fori_loop(0, KB, drain, 0)
                pltpu.sync_copy(nb_hbm.at[l, 1, wid, b], buf)

                def body(j, carry, buf=buf, acc=acc):
                    pltpu.async_copy(ones_v, acc.at[buf.at[j]], sem,
                                     add=True)
                    return carry

                lax.fori_loop(0, KB, body, 0)

        def drain_tail(j, carry):
            pltpu.make_async_copy(ones_v, acc0.at[idx0_v.at[0]], sem).wait()
            return carry

        lax.fori_loop(0, 2 * KB, drain_tail, 0)
        plsc.subcore_barrier()
        pltpu.sync_copy(acc0.at[pl.ds(s * SCH, SCH)],
                        out_hbm.at[c, 0, 0, pl.ds(s * SCH, SCH)])
        pltpu.sync_copy(acc1.at[pl.ds(s * SCH, SCH)],
                        out_hbm.at[c, 1, 0, pl.ds(s * SCH, SCH)])

    return deg_kernel


def _make_edge_kernel(d, layer, tc_tiling=True):
    @functools.partial(
        pl.kernel,
        out_type=jax.ShapeDtypeStruct((NC, NP_, d), jnp.float32),
        mesh=_mesh,
        compiler_params=pltpu.CompilerParams(use_tc_tiling_on_sc=tc_tiling),
        scratch_types=[
            pltpu.VMEM((KB, C), jnp.int32),
            pltpu.VMEM((KB, C), jnp.int32),
            pltpu.VMEM((C, d), jnp.float32),
            pltpu.VMEM((C, d), jnp.float32),
            pltpu.VMEM((C, d), jnp.float32),
            pltpu.VMEM_SHARED((NP_, d), jnp.float32),
            pltpu.SemaphoreType.DMA,
            pltpu.SemaphoreType.DMA,
        ],
    )
    def edge_kernel(nb_hbm, ms_hbm, zeros_hbm, out_hbm,
                    src_v, dst_v, rows0_v, rows1_v, rows2_v, acc,
                    gsem, ssem):
        c = lax.axis_index("c")
        s = lax.axis_index("s")
        wid = c * NS + s
        bufs = (rows0_v, rows1_v, rows2_v)
        pltpu.sync_copy(zeros_hbm.at[pl.ds(s * RP, RP)],
                        acc.at[pl.ds(s * RP, RP)])
        plsc.subcore_barrier()

        def step(cur, nxt, j):
            pltpu.make_async_copy(ms_hbm.at[src_v.at[j]], cur, gsem).wait()

            @pl.when(j >= 2)
            def _():
                pltpu.make_async_copy(nxt, acc.at[dst_v.at[0]], ssem).wait()

            @pl.when(j + 1 < KB)
            def _():
                pltpu.async_copy(ms_hbm.at[src_v.at[j + 1]], nxt, gsem)

            pltpu.async_copy(cur, acc.at[dst_v.at[j]], ssem, add=True)

        def inner(j, carry):
            for r in range(3):
                @pl.when(j % 3 == r)
                def _(r=r):
                    step(bufs[r], bufs[(r + 1) % 3], j)

            return carry

        def block(b, carry):
            pltpu.sync_copy(nb_hbm.at[layer, 0, wid, b], src_v)
            pltpu.sync_copy(nb_hbm.at[layer, 1, wid, b], dst_v)
            pltpu.async_copy(ms_hbm.at[src_v.at[0]], rows0_v, gsem)
            lax.fori_loop(0, KB, inner, 0)

            def drain(j, carry2):
                pltpu.make_async_copy(rows0_v, acc.at[dst_v.at[0]],
                                      ssem).wait()
                return carry2

            lax.fori_loop(0, 2, drain, 0)
            return carry

        lax.fori_loop(0, NB, block, 0)
        plsc.subcore_barrier()
        pltpu.sync_copy(acc.at[pl.ds(s * RP, RP)],
                        out_hbm.at[c, pl.ds(s * RP, RP)])

    return edge_kernel


def _make_edge_kernel_sp(d, layer):
    @functools.partial(
        pl.kernel,
        out_type=jax.ShapeDtypeStruct((NC, NP_, d), jnp.float32),
        mesh=_mesh,
        compiler_params=pltpu.CompilerParams(use_tc_tiling_on_sc=False),
        scratch_types=[
            pltpu.VMEM((KB, C), jnp.int32),
            pltpu.VMEM((KB, C), jnp.int32),
            pltpu.VMEM((C, d), jnp.float32),
            pltpu.VMEM((C, d), jnp.float32),
            pltpu.VMEM((C, d), jnp.float32),
            pltpu.VMEM_SHARED((N, d), jnp.float32),
            pltpu.VMEM_SHARED((NP_, d), jnp.float32),
            pltpu.SemaphoreType.DMA,
            pltpu.SemaphoreType.DMA,
        ],
    )
    def edge_kernel(nb_hbm, ms_hbm, zeros_hbm, out_hbm,
                    src_v, dst_v, rows0_v, rows1_v, rows2_v, ms_sh, acc,
                    gsem, ssem):
        c = lax.axis_index("c")
        s = lax.axis_index("s")
        wid = c * NS + s
        bufs = (rows0_v, rows1_v, rows2_v)
        mrp = N // NS
        pltpu.sync_copy(zeros_hbm.at[pl.ds(s * RP, RP)],
                        acc.at[pl.ds(s * RP, RP)])
        pltpu.sync_copy(ms_hbm.at[pl.ds(s * mrp, mrp)],
                        ms_sh.at[pl.ds(s * mrp, mrp)])
        plsc.subcore_barrier()

        def step(cur, nxt, j):
            pltpu.make_async_copy(ms_sh.at[src_v.at[j]], cur, gsem).wait()

            @pl.when(j >= 2)
            def _():
                pltpu.make_async_copy(nxt, acc.at[dst_v.at[0]], ssem).wait()

            @pl.when(j + 1 < KB)
            def _():
                pltpu.async_copy(ms_sh.at[src_v.at[j + 1]], nxt, gsem)

            pltpu.async_copy(cur, acc.at[dst_v.at[j]], ssem, add=True)

        def inner(j, carry):
            for r in range(3):
                @pl.when(j % 3 == r)
                def _(r=r):
                    step(bufs[r], bufs[(r + 1) % 3], j)

            return carry

        def block(b, carry):
            pltpu.sync_copy(nb_hbm.at[layer, 0, wid, b], src_v)
            pltpu.sync_copy(nb_hbm.at[layer, 1, wid, b], dst_v)
            pltpu.async_copy(ms_sh.at[src_v.at[0]], rows0_v, gsem)
            lax.fori_loop(0, KB, inner, 0)

            def drain(j, carry2):
                pltpu.make_async_copy(rows0_v, acc.at[dst_v.at[0]],
                                      ssem).wait()
                return carry2

            lax.fori_loop(0, 2, drain, 0)
            return carry

        lax.fori_loop(0, NB, block, 0)
        plsc.subcore_barrier()
        pltpu.sync_copy(acc.at[pl.ds(s * RP, RP)],
                        out_hbm.at[c, pl.ds(s * RP, RP)])

    return edge_kernel


_deg_call = _make_deg_kernel()
_edge_call_1 = _make_edge_kernel(DF, 0)
_edge_call_2 = _make_edge_kernel_sp(DC, 1)


def _tc_mm_body(x_ref, w1_ref, m1_ref):
    x = x_ref[...]
    mean = jnp.mean(x, axis=0, keepdims=True)
    var = jnp.mean((x - mean) * (x - mean), axis=0, keepdims=True)
    h = (x - mean) * lax.rsqrt(var + EPS)
    m1_ref[...] = jnp.dot(h, w1_ref[...], preferred_element_type=jnp.float32)


def _tc_mm(x, w1):
    return pl.pallas_call(
        _tc_mm_body,
        out_shape=jax.ShapeDtypeStruct((N, DF), jnp.float32),
    )(x, w1)


def _tc_scale_body(m1_ref, degp_ref, ms1_ref, dinv1_ref, dinv2_ref):
    degp = degp_ref[...]
    deg = degp[0] + degp[1] + 1.0
    dinv = lax.rsqrt(deg)
    d1 = dinv[0, :N]
    d2 = dinv[1, :N]
    ms1_ref[...] = m1_ref[...] * d1[:, None]
    dinv1_ref[...] = d1[:, None]
    dinv2_ref[...] = d2[:, None]


def _tc_scale(m1, degp):
    return pl.pallas_call(
        _tc_scale_body,
        out_shape=[
            jax.ShapeDtypeStruct((N, DF), jnp.float32),
            jax.ShapeDtypeStruct((N, 1), jnp.float32),
            jax.ShapeDtypeStruct((N, 1), jnp.float32),
        ],
    )(m1, degp)


def _tc_mid_body(p1_ref, ms1_ref, dinv1_ref, b1_ref, w2_ref, dinv2_ref,
                 ms2_ref):
    p = p1_ref[0] + p1_ref[1] + ms1_ref[...]
    h1 = jnp.maximum(p * dinv1_ref[...] + b1_ref[...][None, :], 0.0)
    m2 = jnp.dot(h1, w2_ref[...], preferred_element_type=jnp.float32)
    ms2_ref[...] = m2 * dinv2_ref[...]


def _tc_mid(p1, ms1, dinv1, b1, w2, dinv2):
    return pl.pallas_call(
        _tc_mid_body,
        out_shape=jax.ShapeDtypeStruct((N, DC), jnp.float32),
    )(p1, ms1, dinv1, b1, w2, dinv2)


def _tc_final_body(p2_ref, ms2_ref, dinv2_ref, b2_ref, out_ref):
    p = (p2_ref[0] + p2_ref[1] + ms2_ref[...]) * dinv2_ref[...]
    out_ref[...] = p + b2_ref[...][None, :]


def _tc_final(p2, ms2, dinv2, b2):
    return pl.pallas_call(
        _tc_final_body,
        out_shape=jax.ShapeDtypeStruct((N, DC), jnp.float32),
    )(p2, ms2, dinv2, b2)


@jax.jit
def _run(nodeblocks, x, w1, b1, w2, b2):
    nb6 = nodeblocks.astype(jnp.int32).reshape(2, 2, NW, NB, KB, C)

    zeros_deg = jnp.zeros((1, 1, S), jnp.float32)
    zeros_f = jnp.zeros((NP_, DF), jnp.float32)
    zeros_c = jnp.zeros((NP_, DC), jnp.float32)
    ones_c = jnp.ones((C,), jnp.float32)

    m1 = _tc_mm(x, w1)
    degp = _deg_call(nb6, ones_c, zeros_deg)
    ms1, dinv1, dinv2 = _tc_scale(m1, degp.reshape(NC, 2, S))
    p1 = _edge_call_1(nb6, ms1, zeros_f)
    ms2 = _tc_mid(p1[:, :N, :], ms1, dinv1, b1, w2, dinv2)
    p2 = _edge_call_2(nb6, ms2, zeros_c)
    return _tc_final(p2[:, :N, :], ms2, dinv2, b2)


def kernel(nodeblocks, x, W1, b1, W2, b2):
    return _run(nodeblocks, x, W1, b1, W2, b2)


# speedup vs baseline: 38.2715x; 1.0436x over previous
import functools

import jax
import jax.numpy as jnp
from jax import lax
from jax.experimental import pallas as pl
from jax.experimental.pallas import tpu as pltpu
from jax.experimental.pallas import tpu_sc as plsc

N = 10000
E = 320000
DF = 128
DC = 40
EPS = 1e-5

NC, NS = 2, 16
NW = NC * NS
EW = E // NW
C = 80
K = EW // C
KB = 25
NB = K // KB
S = 10240
SCH = S // NS
NP_ = 10112
RP = NP_ // NS

_mesh = plsc.VectorSubcoreMesh(core_axis_name="c", subcore_axis_name="s")


def _make_deg_kernel():
    @functools.partial(
        pl.kernel,
        out_type=jax.ShapeDtypeStruct((NC, 2, 1, S), jnp.float32),
        mesh=_mesh,
        scratch_types=[
            pltpu.VMEM((KB, C), jnp.int32),
            pltpu.VMEM((KB, C), jnp.int32),
            pltpu.VMEM((C,), jnp.float32),
            pltpu.VMEM_SHARED((S,), jnp.float32),
            pltpu.VMEM_SHARED((S,), jnp.float32),
            pltpu.SemaphoreType.DMA,
        ],
    )
    def deg_kernel(nb_hbm, ones_hbm, zeros_hbm, out_hbm,
                   idx0_v, idx1_v, ones_v, acc0, acc1, sem):
        c = lax.axis_index("c")
        s = lax.axis_index("s")
        wid = c * NS + s
        idx_bufs = (idx0_v, idx1_v)
        accs = (acc0, acc1)
        pltpu.sync_copy(zeros_hbm.at[0, 0, pl.ds(s * SCH, SCH)],
                        acc0.at[pl.ds(s * SCH, SCH)])
        pltpu.sync_copy(zeros_hbm.at[0, 0, pl.ds(s * SCH, SCH)],
                        acc1.at[pl.ds(s * SCH, SCH)])
        pltpu.sync_copy(ones_hbm, ones_v)
        plsc.subcore_barrier()

        for l in range(2):
            for b in range(NB):
                k = l * NB + b
                buf = idx_bufs[k % 2]
                acc = accs[l]
                if k >= 2:
                    def drain(j, carry):
                        pltpu.make_async_copy(
                            ones_v, acc0.at[idx0_v.at[0]], sem).wait()
                        return carry

                    lax.fori_loop(0, KB, drain, 0)
                pltpu.sync_copy(nb_hbm.at[l, 1, wid, b], buf)

                def body(j, carry, buf=buf, acc=acc):
                    pltpu.async_copy(ones_v, acc.at[buf.at[j]], sem,
                                     add=True)
                    return carry

                lax.fori_loop(0, KB, body, 0)

        def drain_tail(j, carry):
            pltpu.make_async_copy(ones_v, acc0.at[idx0_v.at[0]], sem).wait()
            return carry

        lax.fori_loop(0, 2 * KB, drain_tail, 0)
        plsc.subcore_barrier()
        pltpu.sync_copy(acc0.at[pl.ds(s * SCH, SCH)],
                        out_hbm.at[c, 0, 0, pl.ds(s * SCH, SCH)])
        pltpu.sync_copy(acc1.at[pl.ds(s * SCH, SCH)],
                        out_hbm.at[c, 1, 0, pl.ds(s * SCH, SCH)])

    return deg_kernel


def _make_edge_kernel(d, layer, tc_tiling=True):
    @functools.partial(
        pl.kernel,
        out_type=jax.ShapeDtypeStruct((NC, NP_, d), jnp.float32),
        mesh=_mesh,
        compiler_params=pltpu.CompilerParams(use_tc_tiling_on_sc=tc_tiling),
        scratch_types=[
            pltpu.VMEM((KB, C), jnp.int32),
            pltpu.VMEM((KB, C), jnp.int32),
            pltpu.VMEM((C, d), jnp.float32),
            pltpu.VMEM((C, d), jnp.float32),
            pltpu.VMEM((C, d), jnp.float32),
            pltpu.VMEM_SHARED((NP_, d), jnp.float32),
            pltpu.SemaphoreType.DMA,
            pltpu.SemaphoreType.DMA,
        ],
    )
    def edge_kernel(nb_hbm, ms_hbm, zeros_hbm, out_hbm,
                    src_v, dst_v, rows0_v, rows1_v, rows2_v, acc,
                    gsem, ssem):
        c = lax.axis_index("c")
        s = lax.axis_index("s")
        wid = c * NS + s
        bufs = (rows0_v, rows1_v, rows2_v)
        pltpu.sync_copy(zeros_hbm.at[pl.ds(s * RP, RP)],
                        acc.at[pl.ds(s * RP, RP)])
        plsc.subcore_barrier()

        def step(cur, nxt, j):
            pltpu.make_async_copy(ms_hbm.at[src_v.at[j]], cur, gsem).wait()

            @pl.when(j >= 2)
            def _():
                pltpu.make_async_copy(nxt, acc.at[dst_v.at[0]], ssem).wait()

            @pl.when(j + 1 < KB)
            def _():
                pltpu.async_copy(ms_hbm.at[src_v.at[j + 1]], nxt, gsem)

            pltpu.async_copy(cur, acc.at[dst_v.at[j]], ssem, add=True)

        def inner(j, carry):
            for r in range(3):
                @pl.when(j % 3 == r)
                def _(r=r):
                    step(bufs[r], bufs[(r + 1) % 3], j)

            return carry

        def block(b, carry):
            pltpu.sync_copy(nb_hbm.at[layer, 0, wid, b], src_v)
            pltpu.sync_copy(nb_hbm.at[layer, 1, wid, b], dst_v)
            pltpu.async_copy(ms_hbm.at[src_v.at[0]], rows0_v, gsem)
            lax.fori_loop(0, KB, inner, 0)

            def drain(j, carry2):
                pltpu.make_async_copy(rows0_v, acc.at[dst_v.at[0]],
                                      ssem).wait()
                return carry2

            lax.fori_loop(0, 2, drain, 0)
            return carry

        lax.fori_loop(0, NB, block, 0)
        plsc.subcore_barrier()
        pltpu.sync_copy(acc.at[pl.ds(s * RP, RP)],
                        out_hbm.at[c, pl.ds(s * RP, RP)])

    return edge_kernel


def _make_edge_kernel_sp(d, layer):
    @functools.partial(
        pl.kernel,
        out_type=jax.ShapeDtypeStruct((NC, NP_, d), jnp.float32),
        mesh=_mesh,
        compiler_params=pltpu.CompilerParams(use_tc_tiling_on_sc=False),
        scratch_types=[
            pltpu.VMEM((KB, C), jnp.int32),
            pltpu.VMEM((KB, C), jnp.int32),
            pltpu.VMEM((C, d), jnp.float32),
            pltpu.VMEM((C, d), jnp.float32),
            pltpu.VMEM((C, d), jnp.float32),
            pltpu.VMEM_SHARED((N, d), jnp.float32),
            pltpu.VMEM_SHARED((NP_, d), jnp.float32),
            pltpu.SemaphoreType.DMA,
            pltpu.SemaphoreType.DMA,
        ],
    )
    def edge_kernel(nb_hbm, ms_hbm, zeros_hbm, out_hbm,
                    src_v, dst_v, rows0_v, rows1_v, rows2_v, ms_sh, acc,
                    gsem, ssem):
        c = lax.axis_index("c")
        s = lax.axis_index("s")
        wid = c * NS + s
        bufs = (rows0_v, rows1_v, rows2_v)
        mrp = N // NS
        pltpu.sync_copy(zeros_hbm.at[pl.ds(s * RP, RP)],
                        acc.at[pl.ds(s * RP, RP)])
        pltpu.sync_copy(ms_hbm.at[pl.ds(s * mrp, mrp)],
                        ms_sh.at[pl.ds(s * mrp, mrp)])
        plsc.subcore_barrier()

        def step(cur, nxt, j):
            pltpu.make_async_copy(ms_sh.at[src_v.at[j]], cur, gsem).wait()

            @pl.when(j >= 2)
            def _():
                pltpu.make_async_copy(nxt, acc.at[dst_v.at[0]], ssem).wait()

            @pl.when(j + 1 < KB)
            def _():
                pltpu.async_copy(ms_sh.at[src_v.at[j + 1]], nxt, gsem)

            pltpu.async_copy(cur, acc.at[dst_v.at[j]], ssem, add=True)

        def inner(j, carry):
            for r in range(3):
                @pl.when(j % 3 == r)
                def _(r=r):
                    step(bufs[r], bufs[(r + 1) % 3], j)

            return carry

        def block(b, carry):
            pltpu.sync_copy(nb_hbm.at[layer, 0, wid, b], src_v)
            pltpu.sync_copy(nb_hbm.at[layer, 1, wid, b], dst_v)
            pltpu.async_copy(ms_sh.at[src_v.at[0]], rows0_v, gsem)
            lax.fori_loop(0, KB, inner, 0)

            def drain(j, carry2):
                pltpu.make_async_copy(rows0_v, acc.at[dst_v.at[0]],
                                      ssem).wait()
                return carry2

            lax.fori_loop(0, 2, drain, 0)
            return carry

        lax.fori_loop(0, NB, block, 0)
        plsc.subcore_barrier()
        pltpu.sync_copy(acc.at[pl.ds(s * RP, RP)],
                        out_hbm.at[c, pl.ds(s * RP, RP)])

    return edge_kernel


_deg_call = _make_deg_kernel()
_edge_call_1 = _make_edge_kernel(DF, 0)
_edge_call_2 = _make_edge_kernel_sp(DC, 1)


def _tc_mm_body(x_ref, w1_ref, m1_ref):
    x = x_ref[...]
    mean = jnp.mean(x, axis=0, keepdims=True)
    var = jnp.mean((x - mean) * (x - mean), axis=0, keepdims=True)
    h = (x - mean) * lax.rsqrt(var + EPS)
    m1_ref[...] = jnp.dot(h, w1_ref[...], preferred_element_type=jnp.float32)


def _tc_mm(x, w1):
    return pl.pallas_call(
        _tc_mm_body,
        out_shape=jax.ShapeDtypeStruct((N, DF), jnp.float32),
    )(x, w1)


def _tc_scale_body(m1_ref, degp_ref, ms1_ref, dinv1_ref, dinv2_ref):
    degp = degp_ref[...]
    deg = degp[0] + degp[1] + 1.0
    dinv = lax.rsqrt(deg)
    d1 = dinv[0, :N]
    d2 = dinv[1, :N]
    ms1_ref[...] = m1_ref[...] * d1[:, None]
    dinv1_ref[...] = d1[:, None]
    dinv2_ref[...] = d2[:, None]


def _tc_scale(m1, degp):
    return pl.pallas_call(
        _tc_scale_body,
        out_shape=[
            jax.ShapeDtypeStruct((N, DF), jnp.float32),
            jax.ShapeDtypeStruct((N, 1), jnp.float32),
            jax.ShapeDtypeStruct((N, 1), jnp.float32),
        ],
    )(m1, degp)


def _tc_mid_body(p1_ref, ms1_ref, dinv1_ref, b1_ref, w2_ref, dinv2_ref,
                 ms2_ref):
    p = p1_ref[0, :N, :] + p1_ref[1, :N, :] + ms1_ref[...]
    h1 = jnp.maximum(p * dinv1_ref[...] + b1_ref[...][None, :], 0.0)
    m2 = jnp.dot(h1, w2_ref[...], preferred_element_type=jnp.float32)
    ms2_ref[...] = m2 * dinv2_ref[...]


def _tc_mid(p1, ms1, dinv1, b1, w2, dinv2):
    return pl.pallas_call(
        _tc_mid_body,
        out_shape=jax.ShapeDtypeStruct((N, DC), jnp.float32),
    )(p1, ms1, dinv1, b1, w2, dinv2)


def _tc_final_body(p2_ref, ms2_ref, dinv2_ref, b2_ref, out_ref):
    p = (p2_ref[0, :N, :] + p2_ref[1, :N, :] + ms2_ref[...]) * dinv2_ref[...]
    out_ref[...] = p + b2_ref[...][None, :]


def _tc_final(p2, ms2, dinv2, b2):
    return pl.pallas_call(
        _tc_final_body,
        out_shape=jax.ShapeDtypeStruct((N, DC), jnp.float32),
    )(p2, ms2, dinv2, b2)


@jax.jit
def _run(nodeblocks, x, w1, b1, w2, b2):
    nb6 = nodeblocks.astype(jnp.int32).reshape(2, 2, NW, NB, KB, C)

    zeros_deg = jnp.zeros((1, 1, S), jnp.float32)
    zeros_f = jnp.zeros((NP_, DF), jnp.float32)
    zeros_c = jnp.zeros((NP_, DC), jnp.float32)
    ones_c = jnp.ones((C,), jnp.float32)

    m1 = _tc_mm(x, w1)
    degp = _deg_call(nb6, ones_c, zeros_deg)
    ms1, dinv1, dinv2 = _tc_scale(m1, degp.reshape(NC, 2, S))
    p1 = _edge_call_1(nb6, ms1, zeros_f)
    ms2 = _tc_mid(p1, ms1, dinv1, b1, w2, dinv2)
    p2 = _edge_call_2(nb6, ms2, zeros_c)
    return _tc_final(p2, ms2, dinv2, b2)


def kernel(nodeblocks, x, W1, b1, W2, b2):
    return _run(nodeblocks, x, W1, b1, W2, b2)


# speedup vs baseline: 38.3265x; 1.0014x over previous
import functools

import jax
import jax.numpy as jnp
from jax import lax
from jax.experimental import pallas as pl
from jax.experimental.pallas import tpu as pltpu
from jax.experimental.pallas import tpu_sc as plsc

N = 10000
E = 320000
DF = 128
DC = 40
EPS = 1e-5

NC, NS = 2, 16
NW = NC * NS
EW = E // NW
C = 80
K = EW // C
KB = 25
NB = K // KB
S = 10240
SCH = S // NS
NP_ = 10112
RP = NP_ // NS

_mesh = plsc.VectorSubcoreMesh(core_axis_name="c", subcore_axis_name="s")


def _make_deg_kernel():
    @functools.partial(
        pl.kernel,
        out_type=jax.ShapeDtypeStruct((NC, 2, 1, S), jnp.float32),
        mesh=_mesh,
        scratch_types=[
            pltpu.VMEM((KB, C), jnp.int32),
            pltpu.VMEM((KB, C), jnp.int32),
            pltpu.VMEM((C,), jnp.float32),
            pltpu.VMEM_SHARED((S,), jnp.float32),
            pltpu.VMEM_SHARED((S,), jnp.float32),
            pltpu.SemaphoreType.DMA,
        ],
    )
    def deg_kernel(nb_hbm, ones_hbm, zeros_hbm, out_hbm,
                   idx0_v, idx1_v, ones_v, acc0, acc1, sem):
        c = lax.axis_index("c")
        s = lax.axis_index("s")
        wid = c * NS + s
        idx_bufs = (idx0_v, idx1_v)
        accs = (acc0, acc1)
        pltpu.sync_copy(zeros_hbm.at[0, 0, pl.ds(s * SCH, SCH)],
                        acc0.at[pl.ds(s * SCH, SCH)])
        pltpu.sync_copy(zeros_hbm.at[0, 0, pl.ds(s * SCH, SCH)],
                        acc1.at[pl.ds(s * SCH, SCH)])
        pltpu.sync_copy(ones_hbm, ones_v)
        plsc.subcore_barrier()

        for l in range(2):
            for b in range(NB):
                k = l * NB + b
                buf = idx_bufs[k % 2]
                acc = accs[l]
                if k >= 2:
                    def drain(j, carry):
                        pltpu.make_async_copy(
                            ones_v, acc0.at[idx0_v.at[0]], sem).wait()
                        return carry

                    lax.fori_loop(0, KB, drain, 0)
                pltpu.sync_copy(nb_hbm.at[l, 1, wid, b], buf)

                def body(j, carry, buf=buf, acc=acc):
                    pltpu.async_copy(ones_v, acc.at[buf.at[j]], sem,
                                     add=True)
                    return carry

                lax.fori_loop(0, KB, body, 0)

        def drain_tail(j, carry):
            pltpu.make_async_copy(ones_v, acc0.at[idx0_v.at[0]], sem).wait()
            return carry

        lax.fori_loop(0, 2 * KB, drain_tail, 0)
        plsc.subcore_barrier()
        pltpu.sync_copy(acc0.at[pl.ds(s * SCH, SCH)],
                        out_hbm.at[c, 0, 0, pl.ds(s * SCH, SCH)])
        pltpu.sync_copy(acc1.at[pl.ds(s * SCH, SCH)],
                        out_hbm.at[c, 1, 0, pl.ds(s * SCH, SCH)])

    return deg_kernel


def _make_edge_kernel(d, layer, tc_tiling=True):
    @functools.partial(
        pl.kernel,
        out_type=jax.ShapeDtypeStruct((NC, NP_, d), jnp.float32),
        mesh=_mesh,
        compiler_params=pltpu.CompilerParams(use_tc_tiling_on_sc=tc_tiling),
        scratch_types=[
            pltpu.VMEM((KB, C), jnp.int32),
            pltpu.VMEM((KB, C), jnp.int32),
            pltpu.VMEM((C, d), jnp.float32),
            pltpu.VMEM((C, d), jnp.float32),
            pltpu.VMEM((C, d), jnp.float32),
            pltpu.VMEM_SHARED((NP_, d), jnp.float32),
            pltpu.SemaphoreType.DMA,
            pltpu.SemaphoreType.DMA,
        ],
    )
    def edge_kernel(nb_hbm, ms_hbm, zeros_hbm, out_hbm,
                    src_v, dst_v, rows0_v, rows1_v, rows2_v, acc,
                    gsem, ssem):
        c = lax.axis_index("c")
        s = lax.axis_index("s")
        wid = c * NS + s
        bufs = (rows0_v, rows1_v, rows2_v)
        pltpu.sync_copy(zeros_hbm.at[pl.ds(s * RP, RP)],
                        acc.at[pl.ds(s * RP, RP)])
        plsc.subcore_barrier()

        def step(cur, nxt, j):
            pltpu.make_async_copy(ms_hbm.at[src_v.at[j]], cur, gsem).wait()

            @pl.when(j >= 2)
            def _():
                pltpu.make_async_copy(nxt, acc.at[dst_v.at[0]], ssem).wait()

            @pl.when(j + 1 < KB)
            def _():
                pltpu.async_copy(ms_hbm.at[src_v.at[j + 1]], nxt, gsem)

            pltpu.async_copy(cur, acc.at[dst_v.at[j]], ssem, add=True)

        def inner(j, carry):
            for r in range(3):
                @pl.when(j % 3 == r)
                def _(r=r):
                    step(bufs[r], bufs[(r + 1) % 3], j)

            return carry

        def block(b, carry):
            pltpu.sync_copy(nb_hbm.at[layer, 0, wid, b], src_v)
            pltpu.sync_copy(nb_hbm.at[layer, 1, wid, b], dst_v)
            pltpu.async_copy(ms_hbm.at[src_v.at[0]], rows0_v, gsem)
            lax.fori_loop(0, KB, inner, 0)

            def drain(j, carry2):
                pltpu.make_async_copy(rows0_v, acc.at[dst_v.at[0]],
                                      ssem).wait()
                return carry2

            lax.fori_loop(0, 2, drain, 0)
            return carry

        lax.fori_loop(0, NB, block, 0)
        plsc.subcore_barrier()
        pltpu.sync_copy(acc.at[pl.ds(s * RP, RP)],
                        out_hbm.at[c, pl.ds(s * RP, RP)])

    return edge_kernel


def _make_edge_kernel_sp(d, layer):
    @functools.partial(
        pl.kernel,
        out_type=jax.ShapeDtypeStruct((NC, NP_, d), jnp.float32),
        mesh=_mesh,
        compiler_params=pltpu.CompilerParams(use_tc_tiling_on_sc=False),
        scratch_types=[
            pltpu.VMEM((KB, C), jnp.int32),
            pltpu.VMEM((KB, C), jnp.int32),
            pltpu.VMEM((C, d), jnp.float32),
            pltpu.VMEM((C, d), jnp.float32),
            pltpu.VMEM((C, d), jnp.float32),
            pltpu.VMEM_SHARED((N, d), jnp.float32),
            pltpu.VMEM_SHARED((NP_, d), jnp.float32),
            pltpu.SemaphoreType.DMA,
            pltpu.SemaphoreType.DMA,
        ],
    )
    def edge_kernel(nb_hbm, ms_hbm, zeros_hbm, out_hbm,
                    src_v, dst_v, rows0_v, rows1_v, rows2_v, ms_sh, acc,
                    gsem, ssem):
        c = lax.axis_index("c")
        s = lax.axis_index("s")
        wid = c * NS + s
        bufs = (rows0_v, rows1_v, rows2_v)
        mrp = N // NS
        pltpu.sync_copy(zeros_hbm.at[pl.ds(s * RP, RP)],
                        acc.at[pl.ds(s * RP, RP)])
        pltpu.sync_copy(ms_hbm.at[pl.ds(s * mrp, mrp)],
                        ms_sh.at[pl.ds(s * mrp, mrp)])
        plsc.subcore_barrier()

        def step(cur, nxt, j):
            pltpu.make_async_copy(ms_sh.at[src_v.at[j]], cur, gsem).wait()

            @pl.when(j >= 2)
            def _():
                pltpu.make_async_copy(nxt, acc.at[dst_v.at[0]], ssem).wait()

            @pl.when(j + 1 < KB)
            def _():
                pltpu.async_copy(ms_sh.at[src_v.at[j + 1]], nxt, gsem)

            pltpu.async_copy(cur, acc.at[dst_v.at[j]], ssem, add=True)

        def inner(j, carry):
            for r in range(3):
                @pl.when(j % 3 == r)
                def _(r=r):
                    step(bufs[r], bufs[(r + 1) % 3], j)

            return carry

        def block(b, carry):
            pltpu.sync_copy(nb_hbm.at[layer, 0, wid, b], src_v)
            pltpu.sync_copy(nb_hbm.at[layer, 1, wid, b], dst_v)
            pltpu.async_copy(ms_sh.at[src_v.at[0]], rows0_v, gsem)
            lax.fori_loop(0, KB, inner, 0)

            def drain(j, carry2):
                pltpu.make_async_copy(rows0_v, acc.at[dst_v.at[0]],
                                      ssem).wait()
                return carry2

            lax.fori_loop(0, 2, drain, 0)
            return carry

        lax.fori_loop(0, NB, block, 0)
        plsc.subcore_barrier()
        pltpu.sync_copy(acc.at[pl.ds(s * RP, RP)],
                        out_hbm.at[c, pl.ds(s * RP, RP)])

    return edge_kernel


_deg_call = _make_deg_kernel()
_edge_call_1 = _make_edge_kernel(DF, 0, tc_tiling=False)
_edge_call_2 = _make_edge_kernel_sp(DC, 1)


def _tc_mm_body(x_ref, w1_ref, m1_ref):
    x = x_ref[...]
    mean = jnp.mean(x, axis=0, keepdims=True)
    var = jnp.mean((x - mean) * (x - mean), axis=0, keepdims=True)
    h = (x - mean) * lax.rsqrt(var + EPS)
    m1_ref[...] = jnp.dot(h, w1_ref[...], preferred_element_type=jnp.float32)


def _tc_mm(x, w1):
    return pl.pallas_call(
        _tc_mm_body,
        out_shape=jax.ShapeDtypeStruct((N, DF), jnp.float32),
    )(x, w1)


def _tc_scale_body(m1_ref, degp_ref, ms1_ref, dinv1_ref, dinv2_ref):
    degp = degp_ref[...]
    deg = degp[0] + degp[1] + 1.0
    dinv = lax.rsqrt(deg)
    d1 = dinv[0, :N]
    d2 = dinv[1, :N]
    ms1_ref[...] = m1_ref[...] * d1[:, None]
    dinv1_ref[...] = d1[:, None]
    dinv2_ref[...] = d2[:, None]


def _tc_scale(m1, degp):
    return pl.pallas_call(
        _tc_scale_body,
        out_shape=[
            jax.ShapeDtypeStruct((N, DF), jnp.float32),
            jax.ShapeDtypeStruct((N, 1), jnp.float32),
            jax.ShapeDtypeStruct((N, 1), jnp.float32),
        ],
    )(m1, degp)


def _tc_mid_body(p1_ref, ms1_ref, dinv1_ref, b1_ref, w2_ref, dinv2_ref,
                 ms2_ref):
    p = p1_ref[0, :N, :] + p1_ref[1, :N, :] + ms1_ref[...]
    h1 = jnp.maximum(p * dinv1_ref[...] + b1_ref[...][None, :], 0.0)
    m2 = jnp.dot(h1, w2_ref[...], preferred_element_type=jnp.float32)
    ms2_ref[...] = m2 * dinv2_ref[...]


def _tc_mid(p1, ms1, dinv1, b1, w2, dinv2):
    return pl.pallas_call(
        _tc_mid_body,
        out_shape=jax.ShapeDtypeStruct((N, DC), jnp.float32),
    )(p1, ms1, dinv1, b1, w2, dinv2)


def _tc_final_body(p2_ref, ms2_ref, dinv2_ref, b2_ref, out_ref):
    p = (p2_ref[0, :N, :] + p2_ref[1, :N, :] + ms2_ref[...]) * dinv2_ref[...]
    out_ref[...] = p + b2_ref[...][None, :]


def _tc_final(p2, ms2, dinv2, b2):
    return pl.pallas_call(
        _tc_final_body,
        out_shape=jax.ShapeDtypeStruct((N, DC), jnp.float32),
    )(p2, ms2, dinv2, b2)


@jax.jit
def _run(nodeblocks, x, w1, b1, w2, b2):
    nb6 = nodeblocks.astype(jnp.int32).reshape(2, 2, NW, NB, KB, C)

    zeros_deg = jnp.zeros((1, 1, S), jnp.float32)
    zeros_f = jnp.zeros((NP_, DF), jnp.float32)
    zeros_c = jnp.zeros((NP_, DC), jnp.float32)
    ones_c = jnp.ones((C,), jnp.float32)

    m1 = _tc_mm(x, w1)
    degp = _deg_call(nb6, ones_c, zeros_deg)
    ms1, dinv1, dinv2 = _tc_scale(m1, degp.reshape(NC, 2, S))
    p1 = _edge_call_1(nb6, ms1, zeros_f)
    ms2 = _tc_mid(p1, ms1, dinv1, b1, w2, dinv2)
    p2 = _edge_call_2(nb6, ms2, zeros_c)
    return _tc_final(p2, ms2, dinv2, b2)


def kernel(nodeblocks, x, W1, b1, W2, b2):
    return _run(nodeblocks, x, W1, b1, W2, b2)


# speedup vs baseline: 38.4453x; 1.0031x over previous
import functools

import jax
import jax.numpy as jnp
from jax import lax
from jax.experimental import pallas as pl
from jax.experimental.pallas import tpu as pltpu
from jax.experimental.pallas import tpu_sc as plsc

N = 10000
E = 320000
DF = 128
DC = 40
EPS = 1e-5

NC, NS = 2, 16
NW = NC * NS
EW = E // NW
C = 80
K = EW // C
KB = 25
NB = K // KB
S = 10240
SCH = S // NS
NP_ = 10112
RP = NP_ // NS

_mesh = plsc.VectorSubcoreMesh(core_axis_name="c", subcore_axis_name="s")


def _make_deg_kernel():
    @functools.partial(
        pl.kernel,
        out_type=jax.ShapeDtypeStruct((NC, 2, 1, S), jnp.float32),
        mesh=_mesh,
        scratch_types=[
            pltpu.VMEM((KB, C), jnp.int32),
            pltpu.VMEM((KB, C), jnp.int32),
            pltpu.VMEM((C,), jnp.float32),
            pltpu.VMEM_SHARED((S,), jnp.float32),
            pltpu.VMEM_SHARED((S,), jnp.float32),
            pltpu.SemaphoreType.DMA,
        ],
    )
    def deg_kernel(nb_hbm, ones_hbm, zeros_hbm, out_hbm,
                   idx0_v, idx1_v, ones_v, acc0, acc1, sem):
        c = lax.axis_index("c")
        s = lax.axis_index("s")
        wid = c * NS + s
        idx_bufs = (idx0_v, idx1_v)
        accs = (acc0, acc1)
        pltpu.sync_copy(zeros_hbm.at[0, 0, pl.ds(s * SCH, SCH)],
                        acc0.at[pl.ds(s * SCH, SCH)])
        pltpu.sync_copy(zeros_hbm.at[0, 0, pl.ds(s * SCH, SCH)],
                        acc1.at[pl.ds(s * SCH, SCH)])
        pltpu.sync_copy(ones_hbm, ones_v)
        plsc.subcore_barrier()

        for l in range(2):
            for b in range(NB):
                k = l * NB + b
                buf = idx_bufs[k % 2]
                acc = accs[l]
                if k >= 2:
                    def drain(j, carry):
                        pltpu.make_async_copy(
                            ones_v, acc0.at[idx0_v.at[0]], sem).wait()
                        return carry

                    lax.fori_loop(0, KB, drain, 0)
                pltpu.sync_copy(nb_hbm.at[l, 1, wid, b], buf)

                def body(j, carry, buf=buf, acc=acc):
                    pltpu.async_copy(ones_v, acc.at[buf.at[j]], sem,
                                     add=True)
                    return carry

                lax.fori_loop(0, KB, body, 0)

        def drain_tail(j, carry):
            pltpu.make_async_copy(ones_v, acc0.at[idx0_v.at[0]], sem).wait()
            return carry

        lax.fori_loop(0, 2 * KB, drain_tail, 0)
        plsc.subcore_barrier()
        pltpu.sync_copy(acc0.at[pl.ds(s * SCH, SCH)],
                        out_hbm.at[c, 0, 0, pl.ds(s * SCH, SCH)])
        pltpu.sync_copy(acc1.at[pl.ds(s * SCH, SCH)],
                        out_hbm.at[c, 1, 0, pl.ds(s * SCH, SCH)])

    return deg_kernel


def _make_edge_kernel(d, layer, tc_tiling=True):
    @functools.partial(
        pl.kernel,
        out_type=jax.ShapeDtypeStruct((NC, NP_, d), jnp.float32),
        mesh=_mesh,
        compiler_params=pltpu.CompilerParams(use_tc_tiling_on_sc=tc_tiling),
        scratch_types=[
            pltpu.VMEM((KB, C), jnp.int32),
            pltpu.VMEM((KB, C), jnp.int32),
            pltpu.VMEM((C, d), jnp.float32),
            pltpu.VMEM((C, d), jnp.float32),
            pltpu.VMEM((C, d), jnp.float32),
            pltpu.VMEM_SHARED((NP_, d), jnp.float32),
            pltpu.SemaphoreType.DMA,
            pltpu.SemaphoreType.DMA,
        ],
    )
    def edge_kernel(nb_hbm, ms_hbm, zeros_hbm, out_hbm,
                    src_v, dst_v, rows0_v, rows1_v, rows2_v, acc,
                    gsem, ssem):
        c = lax.axis_index("c")
        s = lax.axis_index("s")
        wid = c * NS + s
        bufs = (rows0_v, rows1_v, rows2_v)
        pltpu.sync_copy(zeros_hbm.at[pl.ds(s * RP, RP)],
                        acc.at[pl.ds(s * RP, RP)])
        plsc.subcore_barrier()

        def step(cur, nxt, j):
            pltpu.make_async_copy(ms_hbm.at[src_v.at[j]], cur, gsem).wait()

            @pl.when(j >= 2)
            def _():
                pltpu.make_async_copy(nxt, acc.at[dst_v.at[0]], ssem).wait()

            @pl.when(j + 1 < KB)
            def _():
                pltpu.async_copy(ms_hbm.at[src_v.at[j + 1]], nxt, gsem)

            pltpu.async_copy(cur, acc.at[dst_v.at[j]], ssem, add=True)

        def inner(j, carry):
            for r in range(3):
                @pl.when(j % 3 == r)
                def _(r=r):
                    step(bufs[r], bufs[(r + 1) % 3], j)

            return carry

        def block(b, carry):
            pltpu.sync_copy(nb_hbm.at[layer, 0, wid, b], src_v)
            pltpu.sync_copy(nb_hbm.at[layer, 1, wid, b], dst_v)
            pltpu.async_copy(ms_hbm.at[src_v.at[0]], rows0_v, gsem)
            lax.fori_loop(0, KB, inner, 0)

            def drain(j, carry2):
                pltpu.make_async_copy(rows0_v, acc.at[dst_v.at[0]],
                                      ssem).wait()
                return carry2

            lax.fori_loop(0, 2, drain, 0)
            return carry

        lax.fori_loop(0, NB, block, 0)
        plsc.subcore_barrier()
        pltpu.sync_copy(acc.at[pl.ds(s * RP, RP)],
                        out_hbm.at[c, pl.ds(s * RP, RP)])

    return edge_kernel


def _make_edge_kernel_sp(d, layer):
    @functools.partial(
        pl.kernel,
        out_type=jax.ShapeDtypeStruct((NC, NP_, d), jnp.float32),
        mesh=_mesh,
        compiler_params=pltpu.CompilerParams(use_tc_tiling_on_sc=False),
        scratch_types=[
            pltpu.VMEM((KB, C), jnp.int32),
            pltpu.VMEM((KB, C), jnp.int32),
            pltpu.VMEM((C, d), jnp.float32),
            pltpu.VMEM((C, d), jnp.float32),
            pltpu.VMEM((C, d), jnp.float32),
            pltpu.VMEM_SHARED((N, d), jnp.float32),
            pltpu.VMEM_SHARED((NP_, d), jnp.float32),
            pltpu.SemaphoreType.DMA,
            pltpu.SemaphoreType.DMA,
        ],
    )
    def edge_kernel(nb_hbm, ms_hbm, zeros_hbm, out_hbm,
                    src_v, dst_v, rows0_v, rows1_v, rows2_v, ms_sh, acc,
                    gsem, ssem):
        c = lax.axis_index("c")
        s = lax.axis_index("s")
        wid = c * NS + s
        bufs = (rows0_v, rows1_v, rows2_v)
        mrp = N // NS
        pltpu.sync_copy(zeros_hbm.at[pl.ds(s * RP, RP)],
                        acc.at[pl.ds(s * RP, RP)])
        pltpu.sync_copy(ms_hbm.at[pl.ds(s * mrp, mrp)],
                        ms_sh.at[pl.ds(s * mrp, mrp)])
        plsc.subcore_barrier()

        def step(cur, nxt, j):
            pltpu.make_async_copy(ms_sh.at[src_v.at[j]], cur, gsem).wait()

            @pl.when(j >= 2)
            def _():
                pltpu.make_async_copy(nxt, acc.at[dst_v.at[0]], ssem).wait()

            @pl.when(j + 1 < KB)
            def _():
                pltpu.async_copy(ms_sh.at[src_v.at[j + 1]], nxt, gsem)

            pltpu.async_copy(cur, acc.at[dst_v.at[j]], ssem, add=True)

        def inner(j, carry):
            for r in range(3):
                @pl.when(j % 3 == r)
                def _(r=r):
                    step(bufs[r], bufs[(r + 1) % 3], j)

            return carry

        def block(b, carry):
            pltpu.sync_copy(nb_hbm.at[layer, 0, wid, b], src_v)
            pltpu.sync_copy(nb_hbm.at[layer, 1, wid, b], dst_v)
            pltpu.async_copy(ms_sh.at[src_v.at[0]], rows0_v, gsem)
            lax.fori_loop(0, KB, inner, 0)

            def drain(j, carry2):
                pltpu.make_async_copy(rows0_v, acc.at[dst_v.at[0]],
                                      ssem).wait()
                return carry2

            lax.fori_loop(0, 2, drain, 0)
            return carry

        lax.fori_loop(0, NB, block, 0)
        plsc.subcore_barrier()
        pltpu.sync_copy(acc.at[pl.ds(s * RP, RP)],
                        out_hbm.at[c, pl.ds(s * RP, RP)])

    return edge_kernel


_deg_call = _make_deg_kernel()
_edge_call_1 = _make_edge_kernel(DF, 0, tc_tiling=False)
_edge_call_2 = _make_edge_kernel_sp(DC, 1)


def _tc_front_body(x_ref, w1_ref, degp_ref, ms1_ref, dinv1_ref, dinv2_ref):
    x = x_ref[...]
    mean = jnp.mean(x, axis=0, keepdims=True)
    var = jnp.mean((x - mean) * (x - mean), axis=0, keepdims=True)
    h = (x - mean) * lax.rsqrt(var + EPS)
    m1 = jnp.dot(h, w1_ref[...], preferred_element_type=jnp.float32)
    degp = degp_ref[...]
    deg = degp[0] + degp[1] + 1.0
    dinv = lax.rsqrt(deg)
    d1 = dinv[0, :N]
    d2 = dinv[1, :N]
    ms1_ref[...] = m1 * d1[:, None]
    dinv1_ref[...] = d1[:, None]
    dinv2_ref[...] = d2[:, None]


def _tc_front(x, w1, degp):
    return pl.pallas_call(
        _tc_front_body,
        out_shape=[
            jax.ShapeDtypeStruct((N, DF), jnp.float32),
            jax.ShapeDtypeStruct((N, 1), jnp.float32),
            jax.ShapeDtypeStruct((N, 1), jnp.float32),
        ],
    )(x, w1, degp)


def _tc_mid_body(p1_ref, ms1_ref, dinv1_ref, b1_ref, w2_ref, dinv2_ref,
                 ms2_ref):
    p = p1_ref[0, :N, :] + p1_ref[1, :N, :] + ms1_ref[...]
    h1 = jnp.maximum(p * dinv1_ref[...] + b1_ref[...][None, :], 0.0)
    m2 = jnp.dot(h1, w2_ref[...], preferred_element_type=jnp.float32)
    ms2_ref[...] = m2 * dinv2_ref[...]


def _tc_mid(p1, ms1, dinv1, b1, w2, dinv2):
    return pl.pallas_call(
        _tc_mid_body,
        out_shape=jax.ShapeDtypeStruct((N, DC), jnp.float32),
    )(p1, ms1, dinv1, b1, w2, dinv2)


def _tc_final_body(p2_ref, ms2_ref, dinv2_ref, b2_ref, out_ref):
    p = (p2_ref[0, :N, :] + p2_ref[1, :N, :] + ms2_ref[...]) * dinv2_ref[...]
    out_ref[...] = p + b2_ref[...][None, :]


def _tc_final(p2, ms2, dinv2, b2):
    return pl.pallas_call(
        _tc_final_body,
        out_shape=jax.ShapeDtypeStruct((N, DC), jnp.float32),
    )(p2, ms2, dinv2, b2)


@jax.jit
def _run(nodeblocks, x, w1, b1, w2, b2):
    nb6 = nodeblocks.astype(jnp.int32).reshape(2, 2, NW, NB, KB, C)

    zeros_deg = jnp.zeros((1, 1, S), jnp.float32)
    zeros_f = jnp.zeros((NP_, DF), jnp.float32)
    zeros_c = jnp.zeros((NP_, DC), jnp.float32)
    ones_c = jnp.ones((C,), jnp.float32)

    degp = _deg_call(nb6, ones_c, zeros_deg)
    ms1, dinv1, dinv2 = _tc_front(x, w1, degp.reshape(NC, 2, S))
    p1 = _edge_call_1(nb6, ms1, zeros_f)
    ms2 = _tc_mid(p1, ms1, dinv1, b1, w2, dinv2)
    p2 = _edge_call_2(nb6, ms2, zeros_c)
    return _tc_final(p2, ms2, dinv2, b2)


def kernel(nodeblocks, x, W1, b1, W2, b2):
    return _run(nodeblocks, x, W1, b1, W2, b2)


# speedup vs baseline: 38.4717x; 1.0007x over previous
import functools

import jax
import jax.numpy as jnp
from jax import lax
from jax.experimental import pallas as pl
from jax.experimental.pallas import tpu as pltpu
from jax.experimental.pallas import tpu_sc as plsc

N = 10000
E = 320000
DF = 128
DC = 40
EPS = 1e-5

NC, NS = 2, 16
NW = NC * NS
EW = E // NW
C = 80
K = EW // C
KB = 25
NB = K // KB
S = 10240
SCH = S // NS
NP_ = 10112
RP = NP_ // NS

_mesh = plsc.VectorSubcoreMesh(core_axis_name="c", subcore_axis_name="s")


def _make_deg_kernel():
    @functools.partial(
        pl.kernel,
        out_type=jax.ShapeDtypeStruct((NC, 2, 1, S), jnp.float32),
        mesh=_mesh,
        scratch_types=[
            pltpu.VMEM((KB, C), jnp.int32),
            pltpu.VMEM((KB, C), jnp.int32),
            pltpu.VMEM((C,), jnp.float32),
            pltpu.VMEM_SHARED((S,), jnp.float32),
            pltpu.VMEM_SHARED((S,), jnp.float32),
            pltpu.SemaphoreType.DMA,
        ],
    )
    def deg_kernel(nb_hbm, ones_hbm, zeros_hbm, out_hbm,
                   idx0_v, idx1_v, ones_v, acc0, acc1, sem):
        c = lax.axis_index("c")
        s = lax.axis_index("s")
        wid = c * NS + s
        idx_bufs = (idx0_v, idx1_v)
        accs = (acc0, acc1)
        pltpu.sync_copy(zeros_hbm.at[0, 0, pl.ds(s * SCH, SCH)],
                        acc0.at[pl.ds(s * SCH, SCH)])
        pltpu.sync_copy(zeros_hbm.at[0, 0, pl.ds(s * SCH, SCH)],
                        acc1.at[pl.ds(s * SCH, SCH)])
        pltpu.sync_copy(ones_hbm, ones_v)
        plsc.subcore_barrier()

        for l in range(2):
            for b in range(NB):
                k = l * NB + b
                buf = idx_bufs[k % 2]
                acc = accs[l]
                if k >= 2:
                    def drain(j, carry):
                        pltpu.make_async_copy(
                            ones_v, acc0.at[idx0_v.at[0]], sem).wait()
                        return carry

                    lax.fori_loop(0, KB, drain, 0)
                pltpu.sync_copy(nb_hbm.at[l, 1, wid, b], buf)

                def body(j, carry, buf=buf, acc=acc):
                    pltpu.async_copy(ones_v, acc.at[buf.at[j]], sem,
                                     add=True)
                    return carry

                lax.fori_loop(0, KB, body, 0)

        def drain_tail(j, carry):
            pltpu.make_async_copy(ones_v, acc0.at[idx0_v.at[0]], sem).wait()
            return carry

        lax.fori_loop(0, 2 * KB, drain_tail, 0)
        plsc.subcore_barrier()
        pltpu.sync_copy(acc0.at[pl.ds(s * SCH, SCH)],
                        out_hbm.at[c, 0, 0, pl.ds(s * SCH, SCH)])
        pltpu.sync_copy(acc1.at[pl.ds(s * SCH, SCH)],
                        out_hbm.at[c, 1, 0, pl.ds(s * SCH, SCH)])

    return deg_kernel


def _make_edge_kernel(d, layer, tc_tiling=True):
    @functools.partial(
        pl.kernel,
        out_type=jax.ShapeDtypeStruct((NC, NP_, d), jnp.float32),
        mesh=_mesh,
        compiler_params=pltpu.CompilerParams(use_tc_tiling_on_sc=tc_tiling, skip_device_barrier=True),
        scratch_types=[
            pltpu.VMEM((KB, C), jnp.int32),
            pltpu.VMEM((KB, C), jnp.int32),
            pltpu.VMEM((C, d), jnp.float32),
            pltpu.VMEM((C, d), jnp.float32),
            pltpu.VMEM((C, d), jnp.float32),
            pltpu.VMEM_SHARED((NP_, d), jnp.float32),
            pltpu.SemaphoreType.DMA,
            pltpu.SemaphoreType.DMA,
        ],
    )
    def edge_kernel(nb_hbm, ms_hbm, zeros_hbm, out_hbm,
                    src_v, dst_v, rows0_v, rows1_v, rows2_v, acc,
                    gsem, ssem):
        c = lax.axis_index("c")
        s = lax.axis_index("s")
        wid = c * NS + s
        bufs = (rows0_v, rows1_v, rows2_v)
        pltpu.sync_copy(zeros_hbm.at[pl.ds(s * RP, RP)],
                        acc.at[pl.ds(s * RP, RP)])
        plsc.subcore_barrier()

        def step(cur, nxt, j):
            pltpu.make_async_copy(ms_hbm.at[src_v.at[j]], cur, gsem).wait()

            @pl.when(j >= 2)
            def _():
                pltpu.make_async_copy(nxt, acc.at[dst_v.at[0]], ssem).wait()

            @pl.when(j + 1 < KB)
            def _():
                pltpu.async_copy(ms_hbm.at[src_v.at[j + 1]], nxt, gsem)

            pltpu.async_copy(cur, acc.at[dst_v.at[j]], ssem, add=True)

        def inner(j, carry):
            for r in range(3):
                @pl.when(j % 3 == r)
                def _(r=r):
                    step(bufs[r], bufs[(r + 1) % 3], j)

            return carry

        def block(b, carry):
            pltpu.sync_copy(nb_hbm.at[layer, 0, wid, b], src_v)
            pltpu.sync_copy(nb_hbm.at[layer, 1, wid, b], dst_v)
            pltpu.async_copy(ms_hbm.at[src_v.at[0]], rows0_v, gsem)
            lax.fori_loop(0, KB, inner, 0)

            def drain(j, carry2):
                pltpu.make_async_copy(rows0_v, acc.at[dst_v.at[0]],
                                      ssem).wait()
                return carry2

            lax.fori_loop(0, 2, drain, 0)
            return carry

        lax.fori_loop(0, NB, block, 0)
        plsc.subcore_barrier()
        pltpu.sync_copy(acc.at[pl.ds(s * RP, RP)],
                        out_hbm.at[c, pl.ds(s * RP, RP)])

    return edge_kernel


def _make_edge_kernel_sp(d, layer):
    @functools.partial(
        pl.kernel,
        out_type=jax.ShapeDtypeStruct((NC, NP_, d), jnp.float32),
        mesh=_mesh,
        compiler_params=pltpu.CompilerParams(use_tc_tiling_on_sc=False, skip_device_barrier=True),
        scratch_types=[
            pltpu.VMEM((KB, C), jnp.int32),
            pltpu.VMEM((KB, C), jnp.int32),
            pltpu.VMEM((C, d), jnp.float32),
            pltpu.VMEM((C, d), jnp.float32),
            pltpu.VMEM((C, d), jnp.float32),
            pltpu.VMEM_SHARED((N, d), jnp.float32),
            pltpu.VMEM_SHARED((NP_, d), jnp.float32),
            pltpu.SemaphoreType.DMA,
            pltpu.SemaphoreType.DMA,
        ],
    )
    def edge_kernel(nb_hbm, ms_hbm, zeros_hbm, out_hbm,
                    src_v, dst_v, rows0_v, rows1_v, rows2_v, ms_sh, acc,
                    gsem, ssem):
        c = lax.axis_index("c")
        s = lax.axis_index("s")
        wid = c * NS + s
        bufs = (rows0_v, rows1_v, rows2_v)
        mrp = N // NS
        pltpu.sync_copy(zeros_hbm.at[pl.ds(s * RP, RP)],
                        acc.at[pl.ds(s * RP, RP)])
        pltpu.sync_copy(ms_hbm.at[pl.ds(s * mrp, mrp)],
                        ms_sh.at[pl.ds(s * mrp, mrp)])
        plsc.subcore_barrier()

        def step(cur, nxt, j):
            pltpu.make_async_copy(ms_sh.at[src_v.at[j]], cur, gsem).wait()

            @pl.when(j >= 2)
            def _():
                pltpu.make_async_copy(nxt, acc.at[dst_v.at[0]], ssem).wait()

            @pl.when(j + 1 < KB)
            def _():
                pltpu.async_copy(ms_sh.at[src_v.at[j + 1]], nxt, gsem)

            pltpu.async_copy(cur, acc.at[dst_v.at[j]], ssem, add=True)

        def inner(j, carry):
            for r in range(3):
                @pl.when(j % 3 == r)
                def _(r=r):
                    step(bufs[r], bufs[(r + 1) % 3], j)

            return carry

        def block(b, carry):
            pltpu.sync_copy(nb_hbm.at[layer, 0, wid, b], src_v)
            pltpu.sync_copy(nb_hbm.at[layer, 1, wid, b], dst_v)
            pltpu.async_copy(ms_sh.at[src_v.at[0]], rows0_v, gsem)
            lax.fori_loop(0, KB, inner, 0)

            def drain(j, carry2):
                pltpu.make_async_copy(rows0_v, acc.at[dst_v.at[0]],
                                      ssem).wait()
                return carry2

            lax.fori_loop(0, 2, drain, 0)
            return carry

        lax.fori_loop(0, NB, block, 0)
        plsc.subcore_barrier()
        pltpu.sync_copy(acc.at[pl.ds(s * RP, RP)],
                        out_hbm.at[c, pl.ds(s * RP, RP)])

    return edge_kernel


_deg_call = _make_deg_kernel()
_edge_call_1 = _make_edge_kernel(DF, 0, tc_tiling=False)
_edge_call_2 = _make_edge_kernel_sp(DC, 1)


def _tc_front_body(x_ref, w1_ref, degp_ref, ms1_ref, dinv1_ref, dinv2_ref):
    x = x_ref[...]
    mean = jnp.mean(x, axis=0, keepdims=True)
    var = jnp.mean((x - mean) * (x - mean), axis=0, keepdims=True)
    h = (x - mean) * lax.rsqrt(var + EPS)
    m1 = jnp.dot(h, w1_ref[...], preferred_element_type=jnp.float32)
    degp = degp_ref[...]
    deg = degp[0] + degp[1] + 1.0
    dinv = lax.rsqrt(deg)
    d1 = dinv[0, :N]
    d2 = dinv[1, :N]
    ms1_ref[...] = m1 * d1[:, None]
    dinv1_ref[...] = d1[:, None]
    dinv2_ref[...] = d2[:, None]


def _tc_front(x, w1, degp):
    return pl.pallas_call(
        _tc_front_body,
        out_shape=[
            jax.ShapeDtypeStruct((N, DF), jnp.float32),
            jax.ShapeDtypeStruct((N, 1), jnp.float32),
            jax.ShapeDtypeStruct((N, 1), jnp.float32),
        ],
    )(x, w1, degp)


def _tc_mid_body(p1_ref, ms1_ref, dinv1_ref, b1_ref, w2_ref, dinv2_ref,
                 ms2_ref):
    p = p1_ref[0, :N, :] + p1_ref[1, :N, :] + ms1_ref[...]
    h1 = jnp.maximum(p * dinv1_ref[...] + b1_ref[...][None, :], 0.0)
    m2 = jnp.dot(h1, w2_ref[...], preferred_element_type=jnp.float32)
    ms2_ref[...] = m2 * dinv2_ref[...]


def _tc_mid(p1, ms1, dinv1, b1, w2, dinv2):
    return pl.pallas_call(
        _tc_mid_body,
        out_shape=jax.ShapeDtypeStruct((N, DC), jnp.float32),
    )(p1, ms1, dinv1, b1, w2, dinv2)


def _tc_final_body(p2_ref, ms2_ref, dinv2_ref, b2_ref, out_ref):
    p = (p2_ref[0, :N, :] + p2_ref[1, :N, :] + ms2_ref[...]) * dinv2_ref[...]
    out_ref[...] = p + b2_ref[...][None, :]


def _tc_final(p2, ms2, dinv2, b2):
    return pl.pallas_call(
        _tc_final_body,
        out_shape=jax.ShapeDtypeStruct((N, DC), jnp.float32),
    )(p2, ms2, dinv2, b2)


@jax.jit
def _run(nodeblocks, x, w1, b1, w2, b2):
    nb6 = nodeblocks.astype(jnp.int32).reshape(2, 2, NW, NB, KB, C)

    zeros_deg = jnp.zeros((1, 1, S), jnp.float32)
    zeros_f = jnp.zeros((NP_, DF), jnp.float32)
    zeros_c = jnp.zeros((NP_, DC), jnp.float32)
    ones_c = jnp.ones((C,), jnp.float32)

    degp = _deg_call(nb6, ones_c, zeros_deg)
    ms1, dinv1, dinv2 = _tc_front(x, w1, degp.reshape(NC, 2, S))
    p1 = _edge_call_1(nb6, ms1, zeros_f)
    ms2 = _tc_mid(p1, ms1, dinv1, b1, w2, dinv2)
    p2 = _edge_call_2(nb6, ms2, zeros_c)
    return _tc_final(p2, ms2, dinv2, b2)


def kernel(nodeblocks, x, W1, b1, W2, b2):
    return _run(nodeblocks, x, W1, b1, W2, b2)
